# baseline (speedup 1.0000x reference)
import functools

import numpy as np

import jax
import jax.numpy as jnp
from jax import lax
from jax.experimental import pallas as pl
from jax.experimental.pallas import tpu as pltpu
from jax.experimental.pallas import tpu_sc as plsc

D = 128
B = 10
NP = 1000
EP = 32000
N = B * NP
E = B * EP
EPS = 1e-5

NC, NS = 2, 16
NW = NC * NS
EPW = E // NW
CH = 128
NCH = EPW // CH
REM = EPW - NCH * CH
CG = 64
NCG = EPW // CG
REMG = EPW - NCG * CG
NPAD = 10112
RPT = NPAD // NS

TE = 2000
GE = E // TE
TPG = EP // TE

_f32 = jnp.float32
_bf16 = jnp.bfloat16
D2 = D // 2


def _softplus(x):
    return jnp.maximum(x, 0.0) + jnp.log(1.0 + jnp.exp(-jnp.abs(x)))


def _sc_gather(nf, src, dst, zc):
    mesh = plsc.VectorSubcoreMesh(core_axis_name="c", subcore_axis_name="s")

    @functools.partial(
        pl.kernel,
        out_type=(jax.ShapeDtypeStruct((E, D), _f32),
                  jax.ShapeDtypeStruct((E, D), _f32),
                  jax.ShapeDtypeStruct((NC, NPAD, D), _f32)),
        mesh=mesh,
        scratch_types=[
            pltpu.VMEM_SHARED((NPAD, D), _f32),
            pltpu.VMEM((CG,), jnp.int32), pltpu.VMEM((CG,), jnp.int32),
            pltpu.VMEM((CG, D), _f32), pltpu.VMEM((CG, D), _f32),
            pltpu.VMEM((CG, D), _f32),
            pltpu.VMEM((REMG,), jnp.int32), pltpu.VMEM((REMG,), jnp.int32),
            pltpu.VMEM((REMG, D), _f32), pltpu.VMEM((REMG, D), _f32),
            pltpu.SemaphoreType.DMA, pltpu.SemaphoreType.DMA,
        ],
    )
    def k(nf_h, src_h, dst_h, zc_h, gs_h, gd_h, cnt_h,
          csum_sh, sidx, didx, srow, drow, ones,
          sidx2, didx2, srow2, drow2, sem1, sem2):
        c = lax.axis_index("c")
        t = lax.axis_index("s")
        base0 = (c * NS + t) * EPW

        pltpu.sync_copy(zc_h.at[pl.ds(t * RPT, RPT)],
                        csum_sh.at[pl.ds(t * RPT, RPT)])

        def fill(i, carry):
            ones[i // 8, pl.ds((i % 8) * 16, 16)] = jnp.ones((16,), _f32)
            return carry
        lax.fori_loop(0, CG * D // 16, fill, 0)

        plsc.subcore_barrier()

        def chunk(j, carry):
            base = base0 + j * CG
            pltpu.sync_copy(src_h.at[pl.ds(base, CG)], sidx)
            pltpu.sync_copy(dst_h.at[pl.ds(base, CG)], didx)
            cp1 = pltpu.async_copy(nf_h.at[sidx], srow, sem1)
            cp2 = pltpu.async_copy(nf_h.at[didx], drow, sem2)
            pltpu.sync_copy(ones, csum_sh.at[didx], add=True)
            cp1.wait()
            pltpu.sync_copy(srow, gs_h.at[pl.ds(base, CG)])
            cp2.wait()
            pltpu.sync_copy(drow, gd_h.at[pl.ds(base, CG)])
            return carry

        lax.fori_loop(0, NCG, chunk, 0)

        base = base0 + NCG * CG
        pltpu.sync_copy(src_h.at[pl.ds(base, REMG)], sidx2)
        pltpu.sync_copy(dst_h.at[pl.ds(base, REMG)], didx2)
        cp1 = pltpu.async_copy(nf_h.at[sidx2], srow2, sem1)
        cp2 = pltpu.async_copy(nf_h.at[didx2], drow2, sem2)
        pltpu.sync_copy(ones.at[0:REMG], csum_sh.at[didx2], add=True)
        cp1.wait()
        pltpu.sync_copy(srow2, gs_h.at[pl.ds(base, REMG)])
        cp2.wait()
        pltpu.sync_copy(drow2, gd_h.at[pl.ds(base, REMG)])

        plsc.subcore_barrier()
        pltpu.sync_copy(csum_sh.at[pl.ds(t * RPT, RPT)],
                        cnt_h.at[c, pl.ds(t * RPT, RPT)])

    return k(nf, src, dst, zc)


def _sc_scatter(hep, dst, zn):
    mesh = plsc.VectorSubcoreMesh(core_axis_name="c", subcore_axis_name="s")

    @functools.partial(
        pl.kernel,
        out_type=jax.ShapeDtypeStruct((NC, NPAD, D), _f32),
        mesh=mesh,
        scratch_types=[
            pltpu.VMEM_SHARED((NPAD, D), _f32),
            pltpu.VMEM((CH,), jnp.int32),
            pltpu.VMEM((CH, D), _f32),
            pltpu.VMEM((REM,), jnp.int32),
            pltpu.VMEM((REM, D), _f32),
        ],
    )
    def k(hep_h, dst_h, zn_h, parts_h, hsum_sh, didx, rows, didx2, rows2):
        c = lax.axis_index("c")
        t = lax.axis_index("s")

        pltpu.sync_copy(zn_h.at[pl.ds(t * RPT, RPT)],
                        hsum_sh.at[pl.ds(t * RPT, RPT)])
        plsc.subcore_barrier()

        base0 = (c * NS + t) * EPW

        def chunk(j, carry):
            base = base0 + j * CH
            pltpu.sync_copy(dst_h.at[pl.ds(base, CH)], didx)
            pltpu.sync_copy(hep_h.at[pl.ds(base, CH)], rows)
            pltpu.sync_copy(rows, hsum_sh.at[didx], add=True)
            return carry

        lax.fori_loop(0, NCH, chunk, 0)

        base = base0 + NCH * CH
        pltpu.sync_copy(dst_h.at[pl.ds(base, REM)], didx2)
        pltpu.sync_copy(hep_h.at[pl.ds(base, REM)], rows2)
        pltpu.sync_copy(rows2, hsum_sh.at[didx2], add=True)

        plsc.subcore_barrier()

        pltpu.sync_copy(hsum_sh.at[pl.ds(t * RPT, RPT)],
                        parts_h.at[c, pl.ds(t * RPT, RPT)])

    return k(hep, dst, zn)


def _edge1(gs, gd, ef, gf3, w1t, b1):
    def body(gs_r, gd_r, ef_r, gf_r, w_r, b_r, h1_r, st_r):
        i = pl.program_id(0)
        gfb = gf_r[0].astype(_bf16)
        x = jnp.concatenate(
            [gs_r[...].astype(_bf16), gd_r[...].astype(_bf16),
             ef_r[...].astype(_bf16),
             jnp.broadcast_to(gfb, (TE, D))], axis=1)
        h = jnp.dot(x, w_r[...], preferred_element_type=_f32) + b_r[...]
        h1_r[...] = h.astype(jnp.bfloat16)
        s = jnp.sum(h, axis=0, keepdims=True)
        q = jnp.sum(h * h, axis=0, keepdims=True)
        st = jnp.concatenate([s, q], axis=0)

        @pl.when(i == 0)
        def _():
            st_r[...] = jnp.zeros_like(st_r)
        st_r[...] += st

    return pl.pallas_call(
        body,
        grid=(GE,),
        in_specs=[
            pl.BlockSpec((TE, D), lambda i: (i, 0)),
            pl.BlockSpec((TE, D), lambda i: (i, 0)),
            pl.BlockSpec((TE, D), lambda i: (i, 0)),
            pl.BlockSpec((1, 1, D), lambda i: (i // TPG, 0, 0)),
            pl.BlockSpec((4 * D, 4 * D), lambda i: (0, 0)),
            pl.BlockSpec((1, 4 * D), lambda i: (0, 0)),
        ],
        out_specs=[
            pl.BlockSpec((TE, 4 * D), lambda i: (i, 0)),
            pl.BlockSpec((2, 4 * D), lambda i: (0, 0)),
        ],
        out_shape=[jax.ShapeDtypeStruct((E, 4 * D), jnp.bfloat16),
                   jax.ShapeDtypeStruct((2, 4 * D), _f32)],
    )(gs, gd, ef, gf3, w1t, b1)


def _edge2(h1, st1, g1, be1, w2t, b2):
    def body(h1_r, st_r, g_r, be_r, w_r, b_r, h2_r, st2_r):
        i = pl.program_id(0)
        mu = st_r[0:1, :] * (1.0 / E)
        var = st_r[1:2, :] * (1.0 / E) - mu * mu
        sc = g_r[...] * lax.rsqrt(var + EPS)
        sh = be_r[...] - sc * mu
        a = _softplus(h1_r[...].astype(_f32) * sc + sh)
        h2 = jnp.dot(a.astype(_bf16), w_r[...],
                     preferred_element_type=_f32) + b_r[...]
        h2_r[...] = h2
        s = jnp.sum(h2, axis=0, keepdims=True)
        q = jnp.sum(h2 * h2, axis=0, keepdims=True)
        st = jnp.concatenate([s, q], axis=0)

        @pl.when(i == 0)
        def _():
            st2_r[...] = jnp.zeros_like(st2_r)
        st2_r[...] += st

    return pl.pallas_call(
        body,
        grid=(GE,),
        in_specs=[
            pl.BlockSpec((TE, 4 * D), lambda i: (i, 0)),
            pl.BlockSpec((2, 4 * D), lambda i: (0, 0)),
            pl.BlockSpec((1, 4 * D), lambda i: (0, 0)),
            pl.BlockSpec((1, 4 * D), lambda i: (0, 0)),
            pl.BlockSpec((4 * D, D), lambda i: (0, 0)),
            pl.BlockSpec((1, D), lambda i: (0, 0)),
        ],
        out_specs=[
            pl.BlockSpec((TE, D), lambda i: (i, 0)),
            pl.BlockSpec((2, D), lambda i: (0, 0)),
        ],
        out_shape=[jax.ShapeDtypeStruct((E, D), _f32),
                   jax.ShapeDtypeStruct((2, D), _f32)],
    )(h1, st1, g1, be1, w2t, b2)


def _edge3(h2, st2, g2, be2):
    def body(h2_r, st_r, g_r, be_r, he_r):
        mu = st_r[0:1, :] * (1.0 / E)
        var = st_r[1:2, :] * (1.0 / E) - mu * mu
        sc = g_r[...] * lax.rsqrt(var + EPS)
        sh = be_r[...] - sc * mu
        he_r[...] = _softplus(h2_r[...] * sc + sh)

    return pl.pallas_call(
        body,
        grid=(GE,),
        in_specs=[
            pl.BlockSpec((TE, D), lambda i: (i, 0)),
            pl.BlockSpec((2, D), lambda i: (0, 0)),
            pl.BlockSpec((1, D), lambda i: (0, 0)),
            pl.BlockSpec((1, D), lambda i: (0, 0)),
        ],
        out_specs=pl.BlockSpec((TE, D), lambda i: (i, 0)),
        out_shape=jax.ShapeDtypeStruct((E, D), _f32),
    )(h2, st2, g2, be2)


def _node1(nf, parts, cparts, gf3, w3t, b3):
    def body(nf_r, p_r, c_r, gf_r, w_r, b_r, h3_r, st_r, ge_r):
        b = pl.program_id(0)
        pv = p_r[...]
        hs = pv[0] + pv[1]
        cv = c_r[...]
        cnt = (cv[0] + cv[1])[:, 0:1]
        have = hs / jnp.maximum(cnt, 1.0)
        gfb = gf_r[0]
        x = jnp.concatenate(
            [nf_r[...], have, jnp.broadcast_to(gfb, (NP, D))], axis=1)
        h = jnp.dot(x, w_r[...], preferred_element_type=_f32) + b_r[...]
        h3_r[...] = h
        s = jnp.sum(h, axis=0, keepdims=True)
        q = jnp.sum(h * h, axis=0, keepdims=True)
        st = jnp.concatenate([s, q], axis=0)

        @pl.when(b == 0)
        def _():
            st_r[...] = jnp.zeros_like(st_r)
        st_r[...] += st
        ge_r[...] = (jnp.sum(have, axis=0, keepdims=True)
                     * (1.0 / NP))[None]

    return pl.pallas_call(
        body,
        grid=(B,),
        in_specs=[
            pl.BlockSpec((NP, D), lambda b: (b, 0)),
            pl.BlockSpec((NC, NP, D), lambda b: (0, b, 0)),
            pl.BlockSpec((NC, NP, D), lambda b: (0, b, 0)),
            pl.BlockSpec((1, 1, D), lambda b: (b, 0, 0)),
            pl.BlockSpec((3 * D, 3 * D), lambda b: (0, 0)),
            pl.BlockSpec((1, 3 * D), lambda b: (0, 0)),
        ],
        out_specs=[
            pl.BlockSpec((NP, 3 * D), lambda b: (b, 0)),
            pl.BlockSpec((2, 3 * D), lambda b: (0, 0)),
            pl.BlockSpec((1, 1, D), lambda b: (b, 0, 0)),
        ],
        out_shape=[jax.ShapeDtypeStruct((N, 3 * D), _f32),
                   jax.ShapeDtypeStruct((2, 3 * D), _f32),
                   jax.ShapeDtypeStruct((B, 1, D), _f32)],
    )(nf, parts, cparts, gf3, w3t, b3)


def _node2(h3, st3, g3, be3, w4t, b4):
    def body(h3_r, st_r, g_r, be_r, w_r, b_r, h4_r, st4_r):
        b = pl.program_id(0)
        mu = st_r[0:1, :] * (1.0 / N)
        var = st_r[1:2, :] * (1.0 / N) - mu * mu
        sc = g_r[...] * lax.rsqrt(var + EPS)
        sh = be_r[...] - sc * mu
        a = _softplus(h3_r[...] * sc + sh)
        h4 = jnp.dot(a, w_r[...], preferred_element_type=_f32) + b_r[...]
        h4_r[...] = h4
        s = jnp.sum(h4, axis=0, keepdims=True)
        q = jnp.sum(h4 * h4, axis=0, keepdims=True)
        st = jnp.concatenate([s, q], axis=0)

        @pl.when(b == 0)
        def _():
            st4_r[...] = jnp.zeros_like(st4_r)
        st4_r[...] += st

    return pl.pallas_call(
        body,
        grid=(B,),
        in_specs=[
            pl.BlockSpec((NP, 3 * D), lambda b: (b, 0)),
            pl.BlockSpec((2, 3 * D), lambda b: (0, 0)),
            pl.BlockSpec((1, 3 * D), lambda b: (0, 0)),
            pl.BlockSpec((1, 3 * D), lambda b: (0, 0)),
            pl.BlockSpec((3 * D, D), lambda b: (0, 0)),
            pl.BlockSpec((1, D), lambda b: (0, 0)),
        ],
        out_specs=[
            pl.BlockSpec((NP, D), lambda b: (b, 0)),
            pl.BlockSpec((2, D), lambda b: (0, 0)),
        ],
        out_shape=[jax.ShapeDtypeStruct((N, D), _f32),
                   jax.ShapeDtypeStruct((2, D), _f32)],
    )(h3, st3, g3, be3, w4t, b4)


def _node3(h4, st4, g4, be4):
    def body(h4_r, st_r, g_r, be_r, hn_r, gn_r):
        mu = st_r[0:1, :] * (1.0 / N)
        var = st_r[1:2, :] * (1.0 / N) - mu * mu
        sc = g_r[...] * lax.rsqrt(var + EPS)
        sh = be_r[...] - sc * mu
        hn = _softplus(h4_r[...] * sc + sh)
        hn_r[...] = hn
        gn_r[...] = (jnp.sum(hn, axis=0, keepdims=True) * (1.0 / NP))[None]

    return pl.pallas_call(
        body,
        grid=(B,),
        in_specs=[
            pl.BlockSpec((NP, D), lambda b: (b, 0)),
            pl.BlockSpec((2, D), lambda b: (0, 0)),
            pl.BlockSpec((1, D), lambda b: (0, 0)),
            pl.BlockSpec((1, D), lambda b: (0, 0)),
        ],
        out_specs=[
            pl.BlockSpec((NP, D), lambda b: (b, 0)),
            pl.BlockSpec((1, 1, D), lambda b: (b, 0, 0)),
        ],
        out_shape=[jax.ShapeDtypeStruct((N, D), _f32),
                   jax.ShapeDtypeStruct((B, 1, D), _f32)],
    )(h4, st4, g4, be4)


def _glob(gn, ge, gf, wg1t, bg1, gg1, beg1, wg2t, bg2, gg2, beg2):
    def bn(h, g, be):
        mu = jnp.mean(h, axis=0, keepdims=True)
        var = jnp.mean(h * h, axis=0, keepdims=True) - mu * mu
        return _softplus(g * ((h - mu) * lax.rsqrt(var + EPS)) + be)

    def body(gn_r, ge_r, gf_r, w1_r, b1_r, g1_r, be1_r,
             w2_r, b2_r, g2_r, be2_r, hg_r):
        x = jnp.concatenate([gn_r[...], ge_r[...], gf_r[...]], axis=1)
        h = jnp.dot(x, w1_r[...], preferred_element_type=_f32) + b1_r[...]
        h = bn(h, g1_r[...], be1_r[...])
        h2 = jnp.dot(h, w2_r[...], preferred_element_type=_f32) + b2_r[...]
        hg_r[...] = bn(h2, g2_r[...], be2_r[...])

    return pl.pallas_call(
        body,
        out_shape=jax.ShapeDtypeStruct((B, D), _f32),
    )(gn, ge, gf, wg1t, bg1, gg1, beg1, wg2t, bg2, gg2, beg2)


def kernel(node_feats, edge_feats, global_feats, params, src, dst,
           node_gid, batch_num_nodes, batch_num_edges):
    p = params
    r1 = lambda v: v.reshape(1, -1)
    gf3 = global_feats.reshape(B, 1, D)

    gs, gd, cparts = _sc_gather(node_feats, src, dst,
                                jnp.zeros((NPAD, D), _f32))
    h1, st1 = _edge1(gs, gd, edge_feats, gf3,
                     p['e1_W'].T.astype(_bf16), r1(p['e1_b']))
    h2, st2 = _edge2(h1, st1, r1(p['e1_g']), r1(p['e1_be']),
                     p['e2_W'].T.astype(_bf16), r1(p['e2_b']))
    he = _edge3(h2, st2, r1(p['e2_g']), r1(p['e2_be']))

    parts = _sc_scatter(he, dst, jnp.zeros((NPAD, D), _f32))

    h3, st3, gedge = _node1(node_feats, parts, cparts, gf3,
                            p['n1_W'].T, r1(p['n1_b']))
    h4, st4 = _node2(h3, st3, r1(p['n1_g']), r1(p['n1_be']),
                     p['n2_W'].T, r1(p['n2_b']))
    hn, gnode = _node3(h4, st4, r1(p['n2_g']), r1(p['n2_be']))

    hg = _glob(gnode.reshape(B, D), gedge.reshape(B, D), global_feats,
               p['g1_W'].T, r1(p['g1_b']), r1(p['g1_g']), r1(p['g1_be']),
               p['g2_W'].T, r1(p['g2_b']), r1(p['g2_g']), r1(p['g2_be']))

    return hn, he, hg


# speedup vs baseline: 1.1273x; 1.1273x over previous
import functools

import numpy as np

import jax
import jax.numpy as jnp
from jax import lax
from jax.experimental import pallas as pl
from jax.experimental.pallas import tpu as pltpu
from jax.experimental.pallas import tpu_sc as plsc

D = 128
B = 10
NP = 1000
EP = 32000
N = B * NP
E = B * EP
EPS = 1e-5

NC, NS = 2, 16
NW = NC * NS
EPW = E // NW
CH = 128
NCH = EPW // CH
REM = EPW - NCH * CH
CG = 64
NCG = EPW // CG
REMG = EPW - NCG * CG
NPAD = 10112
RPT = NPAD // NS

TE = 2000
GE = E // TE
TPG = EP // TE

_f32 = jnp.float32
_bf16 = jnp.bfloat16
D2 = D // 2


def _softplus(x):
    return jnp.maximum(x, 0.0) + jnp.log(1.0 + jnp.exp(-jnp.abs(x)))


def _sc_gather(nf, src, dst, zc):
    mesh = plsc.VectorSubcoreMesh(core_axis_name="c", subcore_axis_name="s")

    @functools.partial(
        pl.kernel,
        out_type=(jax.ShapeDtypeStruct((E, D), _f32),
                  jax.ShapeDtypeStruct((E, D), _f32),
                  jax.ShapeDtypeStruct((NC, NPAD, D), _f32)),
        mesh=mesh,
        scratch_types=[
            pltpu.VMEM_SHARED((NPAD, D), _f32),
            pltpu.VMEM((CG,), jnp.int32), pltpu.VMEM((CG,), jnp.int32),
            pltpu.VMEM((CG,), jnp.int32), pltpu.VMEM((CG,), jnp.int32),
            pltpu.VMEM((CG, D), _f32), pltpu.VMEM((CG, D), _f32),
            pltpu.VMEM((CG, D), _f32), pltpu.VMEM((CG, D), _f32),
            pltpu.VMEM((CG, D), _f32),
            pltpu.VMEM((REMG,), jnp.int32), pltpu.VMEM((REMG,), jnp.int32),
            pltpu.VMEM((REMG, D), _f32), pltpu.VMEM((REMG, D), _f32),
            pltpu.SemaphoreType.DMA, pltpu.SemaphoreType.DMA,
            pltpu.SemaphoreType.DMA, pltpu.SemaphoreType.DMA,
            pltpu.SemaphoreType.DMA, pltpu.SemaphoreType.DMA,
        ],
    )
    def k(nf_h, src_h, dst_h, zc_h, gs_h, gd_h, cnt_h,
          csum_sh, sA, dA, sB, dB, srA, drA, srB, drB, ones,
          sidx2, didx2, srow2, drow2,
          semIA, semIB, semGA, semGB, semCA, semCB):
        c = lax.axis_index("c")
        t = lax.axis_index("s")
        base0 = (c * NS + t) * EPW

        pltpu.sync_copy(zc_h.at[pl.ds(t * RPT, RPT)],
                        csum_sh.at[pl.ds(t * RPT, RPT)])

        def fill(i, carry):
            ones[i // 8, pl.ds((i % 8) * 16, 16)] = jnp.ones((16,), _f32)
            return carry
        lax.fori_loop(0, CG * D // 16, fill, 0)

        plsc.subcore_barrier()

        pltpu.async_copy(src_h.at[pl.ds(base0, CG)], sA, semIA)
        pltpu.async_copy(dst_h.at[pl.ds(base0, CG)], dA, semIA)

        npair = NCG // 2

        def pair(m, carry):
            baseA = base0 + (2 * m) * CG
            baseB = baseA + CG
            baseN = jnp.where(m == npair - 1,
                              base0, base0 + (2 * m + 2) * CG)
            pltpu.make_async_copy(src_h.at[pl.ds(baseA, CG)], sA, semIA).wait()
            pltpu.make_async_copy(dst_h.at[pl.ds(baseA, CG)], dA, semIA).wait()
            cpA1 = pltpu.async_copy(nf_h.at[sA], srA, semGA)
            cpA2 = pltpu.async_copy(nf_h.at[dA], drA, semGA)
            pltpu.async_copy(src_h.at[pl.ds(baseB, CG)], sB, semIB)
            pltpu.async_copy(dst_h.at[pl.ds(baseB, CG)], dB, semIB)
            ccA = pltpu.async_copy(ones, csum_sh.at[dA], semCA, add=True)
            pltpu.make_async_copy(src_h.at[pl.ds(baseB, CG)], sB, semIB).wait()
            pltpu.make_async_copy(dst_h.at[pl.ds(baseB, CG)], dB, semIB).wait()
            cpB1 = pltpu.async_copy(nf_h.at[sB], srB, semGB)
            cpB2 = pltpu.async_copy(nf_h.at[dB], drB, semGB)
            ccB = pltpu.async_copy(ones, csum_sh.at[dB], semCB, add=True)
            cpA1.wait()
            pltpu.sync_copy(srA, gs_h.at[pl.ds(baseA, CG)])
            cpA2.wait()
            pltpu.sync_copy(drA, gd_h.at[pl.ds(baseA, CG)])
            cpB1.wait()
            pltpu.sync_copy(srB, gs_h.at[pl.ds(baseB, CG)])
            cpB2.wait()
            pltpu.sync_copy(drB, gd_h.at[pl.ds(baseB, CG)])
            ccA.wait()
            ccB.wait()
            pltpu.async_copy(src_h.at[pl.ds(baseN, CG)], sA, semIA)
            pltpu.async_copy(dst_h.at[pl.ds(baseN, CG)], dA, semIA)
            return carry

        lax.fori_loop(0, npair, pair, 0)

        pltpu.make_async_copy(src_h.at[pl.ds(base0, CG)], sA, semIA).wait()
        pltpu.make_async_copy(dst_h.at[pl.ds(base0, CG)], dA, semIA).wait()

        base = base0 + NCG * CG
        pltpu.sync_copy(src_h.at[pl.ds(base, REMG)], sidx2)
        pltpu.sync_copy(dst_h.at[pl.ds(base, REMG)], didx2)
        cp1 = pltpu.async_copy(nf_h.at[sidx2], srow2, semGA)
        cp2 = pltpu.async_copy(nf_h.at[didx2], drow2, semGB)
        pltpu.sync_copy(ones.at[0:REMG], csum_sh.at[didx2], add=True)
        cp1.wait()
        pltpu.sync_copy(srow2, gs_h.at[pl.ds(base, REMG)])
        cp2.wait()
        pltpu.sync_copy(drow2, gd_h.at[pl.ds(base, REMG)])

        plsc.subcore_barrier()
        pltpu.sync_copy(csum_sh.at[pl.ds(t * RPT, RPT)],
                        cnt_h.at[c, pl.ds(t * RPT, RPT)])

    return k(nf, src, dst, zc)


def _sc_scatter(hep, dst, zn):
    mesh = plsc.VectorSubcoreMesh(core_axis_name="c", subcore_axis_name="s")

    @functools.partial(
        pl.kernel,
        out_type=jax.ShapeDtypeStruct((NC, NPAD, D), _f32),
        mesh=mesh,
        scratch_types=[
            pltpu.VMEM_SHARED((NPAD, D), _f32),
            pltpu.VMEM((CH,), jnp.int32),
            pltpu.VMEM((CH, D), _f32),
            pltpu.VMEM((REM,), jnp.int32),
            pltpu.VMEM((REM, D), _f32),
        ],
    )
    def k(hep_h, dst_h, zn_h, parts_h, hsum_sh, didx, rows, didx2, rows2):
        c = lax.axis_index("c")
        t = lax.axis_index("s")

        pltpu.sync_copy(zn_h.at[pl.ds(t * RPT, RPT)],
                        hsum_sh.at[pl.ds(t * RPT, RPT)])
        plsc.subcore_barrier()

        base0 = (c * NS + t) * EPW

        def chunk(j, carry):
            base = base0 + j * CH
            pltpu.sync_copy(dst_h.at[pl.ds(base, CH)], didx)
            pltpu.sync_copy(hep_h.at[pl.ds(base, CH)], rows)
            pltpu.sync_copy(rows, hsum_sh.at[didx], add=True)
            return carry

        lax.fori_loop(0, NCH, chunk, 0)

        base = base0 + NCH * CH
        pltpu.sync_copy(dst_h.at[pl.ds(base, REM)], didx2)
        pltpu.sync_copy(hep_h.at[pl.ds(base, REM)], rows2)
        pltpu.sync_copy(rows2, hsum_sh.at[didx2], add=True)

        plsc.subcore_barrier()

        pltpu.sync_copy(hsum_sh.at[pl.ds(t * RPT, RPT)],
                        parts_h.at[c, pl.ds(t * RPT, RPT)])

    return k(hep, dst, zn)


def _edge1(gs, gd, ef, gf3, w1t, b1):
    def body(gs_r, gd_r, ef_r, gf_r, w_r, b_r, h1_r, st_r):
        i = pl.program_id(0)
        gfb = gf_r[0].astype(_bf16)
        x = jnp.concatenate(
            [gs_r[...].astype(_bf16), gd_r[...].astype(_bf16),
             ef_r[...].astype(_bf16),
             jnp.broadcast_to(gfb, (TE, D))], axis=1)
        h = jnp.dot(x, w_r[...], preferred_element_type=_f32) + b_r[...]
        h1_r[...] = h.astype(jnp.bfloat16)
        s = jnp.sum(h, axis=0, keepdims=True)
        q = jnp.sum(h * h, axis=0, keepdims=True)
        st = jnp.concatenate([s, q], axis=0)

        @pl.when(i == 0)
        def _():
            st_r[...] = jnp.zeros_like(st_r)
        st_r[...] += st

    return pl.pallas_call(
        body,
        grid=(GE,),
        in_specs=[
            pl.BlockSpec((TE, D), lambda i: (i, 0)),
            pl.BlockSpec((TE, D), lambda i: (i, 0)),
            pl.BlockSpec((TE, D), lambda i: (i, 0)),
            pl.BlockSpec((1, 1, D), lambda i: (i // TPG, 0, 0)),
            pl.BlockSpec((4 * D, 4 * D), lambda i: (0, 0)),
            pl.BlockSpec((1, 4 * D), lambda i: (0, 0)),
        ],
        out_specs=[
            pl.BlockSpec((TE, 4 * D), lambda i: (i, 0)),
            pl.BlockSpec((2, 4 * D), lambda i: (0, 0)),
        ],
        out_shape=[jax.ShapeDtypeStruct((E, 4 * D), jnp.bfloat16),
                   jax.ShapeDtypeStruct((2, 4 * D), _f32)],
    )(gs, gd, ef, gf3, w1t, b1)


def _edge2(h1, st1, g1, be1, w2t, b2):
    def body(h1_r, st_r, g_r, be_r, w_r, b_r, h2_r, st2_r):
        i = pl.program_id(0)
        mu = st_r[0:1, :] * (1.0 / E)
        var = st_r[1:2, :] * (1.0 / E) - mu * mu
        sc = g_r[...] * lax.rsqrt(var + EPS)
        sh = be_r[...] - sc * mu
        a = _softplus(h1_r[...].astype(_f32) * sc + sh)
        h2 = jnp.dot(a.astype(_bf16), w_r[...],
                     preferred_element_type=_f32) + b_r[...]
        h2_r[...] = h2
        s = jnp.sum(h2, axis=0, keepdims=True)
        q = jnp.sum(h2 * h2, axis=0, keepdims=True)
        st = jnp.concatenate([s, q], axis=0)

        @pl.when(i == 0)
        def _():
            st2_r[...] = jnp.zeros_like(st2_r)
        st2_r[...] += st

    return pl.pallas_call(
        body,
        grid=(GE,),
        in_specs=[
            pl.BlockSpec((TE, 4 * D), lambda i: (i, 0)),
            pl.BlockSpec((2, 4 * D), lambda i: (0, 0)),
            pl.BlockSpec((1, 4 * D), lambda i: (0, 0)),
            pl.BlockSpec((1, 4 * D), lambda i: (0, 0)),
            pl.BlockSpec((4 * D, D), lambda i: (0, 0)),
            pl.BlockSpec((1, D), lambda i: (0, 0)),
        ],
        out_specs=[
            pl.BlockSpec((TE, D), lambda i: (i, 0)),
            pl.BlockSpec((2, D), lambda i: (0, 0)),
        ],
        out_shape=[jax.ShapeDtypeStruct((E, D), _f32),
                   jax.ShapeDtypeStruct((2, D), _f32)],
    )(h1, st1, g1, be1, w2t, b2)


def _edge3(h2, st2, g2, be2):
    def body(h2_r, st_r, g_r, be_r, he_r):
        mu = st_r[0:1, :] * (1.0 / E)
        var = st_r[1:2, :] * (1.0 / E) - mu * mu
        sc = g_r[...] * lax.rsqrt(var + EPS)
        sh = be_r[...] - sc * mu
        he_r[...] = _softplus(h2_r[...] * sc + sh)

    return pl.pallas_call(
        body,
        grid=(GE,),
        in_specs=[
            pl.BlockSpec((TE, D), lambda i: (i, 0)),
            pl.BlockSpec((2, D), lambda i: (0, 0)),
            pl.BlockSpec((1, D), lambda i: (0, 0)),
            pl.BlockSpec((1, D), lambda i: (0, 0)),
        ],
        out_specs=pl.BlockSpec((TE, D), lambda i: (i, 0)),
        out_shape=jax.ShapeDtypeStruct((E, D), _f32),
    )(h2, st2, g2, be2)


def _node1(nf, parts, cparts, gf3, w3t, b3):
    def body(nf_r, p_r, c_r, gf_r, w_r, b_r, h3_r, st_r, ge_r):
        b = pl.program_id(0)
        pv = p_r[...]
        hs = pv[0] + pv[1]
        cv = c_r[...]
        cnt = (cv[0] + cv[1])[:, 0:1]
        have = hs / jnp.maximum(cnt, 1.0)
        gfb = gf_r[0]
        x = jnp.concatenate(
            [nf_r[...], have, jnp.broadcast_to(gfb, (NP, D))], axis=1)
        h = jnp.dot(x, w_r[...], preferred_element_type=_f32) + b_r[...]
        h3_r[...] = h
        s = jnp.sum(h, axis=0, keepdims=True)
        q = jnp.sum(h * h, axis=0, keepdims=True)
        st = jnp.concatenate([s, q], axis=0)

        @pl.when(b == 0)
        def _():
            st_r[...] = jnp.zeros_like(st_r)
        st_r[...] += st
        ge_r[...] = (jnp.sum(have, axis=0, keepdims=True)
                     * (1.0 / NP))[None]

    return pl.pallas_call(
        body,
        grid=(B,),
        in_specs=[
            pl.BlockSpec((NP, D), lambda b: (b, 0)),
            pl.BlockSpec((NC, NP, D), lambda b: (0, b, 0)),
            pl.BlockSpec((NC, NP, D), lambda b: (0, b, 0)),
            pl.BlockSpec((1, 1, D), lambda b: (b, 0, 0)),
            pl.BlockSpec((3 * D, 3 * D), lambda b: (0, 0)),
            pl.BlockSpec((1, 3 * D), lambda b: (0, 0)),
        ],
        out_specs=[
            pl.BlockSpec((NP, 3 * D), lambda b: (b, 0)),
            pl.BlockSpec((2, 3 * D), lambda b: (0, 0)),
            pl.BlockSpec((1, 1, D), lambda b: (b, 0, 0)),
        ],
        out_shape=[jax.ShapeDtypeStruct((N, 3 * D), _f32),
                   jax.ShapeDtypeStruct((2, 3 * D), _f32),
                   jax.ShapeDtypeStruct((B, 1, D), _f32)],
    )(nf, parts, cparts, gf3, w3t, b3)


def _node2(h3, st3, g3, be3, w4t, b4):
    def body(h3_r, st_r, g_r, be_r, w_r, b_r, h4_r, st4_r):
        b = pl.program_id(0)
        mu = st_r[0:1, :] * (1.0 / N)
        var = st_r[1:2, :] * (1.0 / N) - mu * mu
        sc = g_r[...] * lax.rsqrt(var + EPS)
        sh = be_r[...] - sc * mu
        a = _softplus(h3_r[...] * sc + sh)
        h4 = jnp.dot(a, w_r[...], preferred_element_type=_f32) + b_r[...]
        h4_r[...] = h4
        s = jnp.sum(h4, axis=0, keepdims=True)
        q = jnp.sum(h4 * h4, axis=0, keepdims=True)
        st = jnp.concatenate([s, q], axis=0)

        @pl.when(b == 0)
        def _():
            st4_r[...] = jnp.zeros_like(st4_r)
        st4_r[...] += st

    return pl.pallas_call(
        body,
        grid=(B,),
        in_specs=[
            pl.BlockSpec((NP, 3 * D), lambda b: (b, 0)),
            pl.BlockSpec((2, 3 * D), lambda b: (0, 0)),
            pl.BlockSpec((1, 3 * D), lambda b: (0, 0)),
            pl.BlockSpec((1, 3 * D), lambda b: (0, 0)),
            pl.BlockSpec((3 * D, D), lambda b: (0, 0)),
            pl.BlockSpec((1, D), lambda b: (0, 0)),
        ],
        out_specs=[
            pl.BlockSpec((NP, D), lambda b: (b, 0)),
            pl.BlockSpec((2, D), lambda b: (0, 0)),
        ],
        out_shape=[jax.ShapeDtypeStruct((N, D), _f32),
                   jax.ShapeDtypeStruct((2, D), _f32)],
    )(h3, st3, g3, be3, w4t, b4)


def _node3(h4, st4, g4, be4):
    def body(h4_r, st_r, g_r, be_r, hn_r, gn_r):
        mu = st_r[0:1, :] * (1.0 / N)
        var = st_r[1:2, :] * (1.0 / N) - mu * mu
        sc = g_r[...] * lax.rsqrt(var + EPS)
        sh = be_r[...] - sc * mu
        hn = _softplus(h4_r[...] * sc + sh)
        hn_r[...] = hn
        gn_r[...] = (jnp.sum(hn, axis=0, keepdims=True) * (1.0 / NP))[None]

    return pl.pallas_call(
        body,
        grid=(B,),
        in_specs=[
            pl.BlockSpec((NP, D), lambda b: (b, 0)),
            pl.BlockSpec((2, D), lambda b: (0, 0)),
            pl.BlockSpec((1, D), lambda b: (0, 0)),
            pl.BlockSpec((1, D), lambda b: (0, 0)),
        ],
        out_specs=[
            pl.BlockSpec((NP, D), lambda b: (b, 0)),
            pl.BlockSpec((1, 1, D), lambda b: (b, 0, 0)),
        ],
        out_shape=[jax.ShapeDtypeStruct((N, D), _f32),
                   jax.ShapeDtypeStruct((B, 1, D), _f32)],
    )(h4, st4, g4, be4)


def _glob(gn, ge, gf, wg1t, bg1, gg1, beg1, wg2t, bg2, gg2, beg2):
    def bn(h, g, be):
        mu = jnp.mean(h, axis=0, keepdims=True)
        var = jnp.mean(h * h, axis=0, keepdims=True) - mu * mu
        return _softplus(g * ((h - mu) * lax.rsqrt(var + EPS)) + be)

    def body(gn_r, ge_r, gf_r, w1_r, b1_r, g1_r, be1_r,
             w2_r, b2_r, g2_r, be2_r, hg_r):
        x = jnp.concatenate([gn_r[...], ge_r[...], gf_r[...]], axis=1)
        h = jnp.dot(x, w1_r[...], preferred_element_type=_f32) + b1_r[...]
        h = bn(h, g1_r[...], be1_r[...])
        h2 = jnp.dot(h, w2_r[...], preferred_element_type=_f32) + b2_r[...]
        hg_r[...] = bn(h2, g2_r[...], be2_r[...])

    return pl.pallas_call(
        body,
        out_shape=jax.ShapeDtypeStruct((B, D), _f32),
    )(gn, ge, gf, wg1t, bg1, gg1, beg1, wg2t, bg2, gg2, beg2)


def kernel(node_feats, edge_feats, global_feats, params, src, dst,
           node_gid, batch_num_nodes, batch_num_edges):
    p = params
    r1 = lambda v: v.reshape(1, -1)
    gf3 = global_feats.reshape(B, 1, D)

    gs, gd, cparts = _sc_gather(node_feats, src, dst,
                                jnp.zeros((NPAD, D), _f32))
    h1, st1 = _edge1(gs, gd, edge_feats, gf3,
                     p['e1_W'].T.astype(_bf16), r1(p['e1_b']))
    h2, st2 = _edge2(h1, st1, r1(p['e1_g']), r1(p['e1_be']),
                     p['e2_W'].T.astype(_bf16), r1(p['e2_b']))
    he = _edge3(h2, st2, r1(p['e2_g']), r1(p['e2_be']))

    parts = _sc_scatter(he, dst, jnp.zeros((NPAD, D), _f32))

    h3, st3, gedge = _node1(node_feats, parts, cparts, gf3,
                            p['n1_W'].T, r1(p['n1_b']))
    h4, st4 = _node2(h3, st3, r1(p['n1_g']), r1(p['n1_be']),
                     p['n2_W'].T, r1(p['n2_b']))
    hn, gnode = _node3(h4, st4, r1(p['n2_g']), r1(p['n2_be']))

    hg = _glob(gnode.reshape(B, D), gedge.reshape(B, D), global_feats,
               p['g1_W'].T, r1(p['g1_b']), r1(p['g1_g']), r1(p['g1_be']),
               p['g2_W'].T, r1(p['g2_b']), r1(p['g2_g']), r1(p['g2_be']))

    return hn, he, hg


# speedup vs baseline: 1.1979x; 1.0626x over previous
import functools

import numpy as np

import jax
import jax.numpy as jnp
from jax import lax
from jax.experimental import pallas as pl
from jax.experimental.pallas import tpu as pltpu
from jax.experimental.pallas import tpu_sc as plsc

D = 128
B = 10
NP = 1000
EP = 32000
N = B * NP
E = B * EP
EPS = 1e-5

NC, NS = 2, 16
NW = NC * NS
EPW = E // NW
CH = 128
NCH = EPW // CH
REM = EPW - NCH * CH
CG = 64
NCG = EPW // CG
REMG = EPW - NCG * CG
NPAD = 10112
RPT = NPAD // NS

TE = 2000
GE = E // TE
TPG = EP // TE

_f32 = jnp.float32
_bf16 = jnp.bfloat16
D2 = D // 2


def _softplus(x):
    return jnp.maximum(x, 0.0) + jnp.log(1.0 + jnp.exp(-jnp.abs(x)))


def _sc_gather(nf, src, dst, zc):
    mesh = plsc.VectorSubcoreMesh(core_axis_name="c", subcore_axis_name="s")

    @functools.partial(
        pl.kernel,
        out_type=(jax.ShapeDtypeStruct((E, D), _f32),
                  jax.ShapeDtypeStruct((E, D), _f32),
                  jax.ShapeDtypeStruct((NC, NPAD, D), _f32)),
        mesh=mesh,
        scratch_types=[
            pltpu.VMEM_SHARED((NPAD, D), _f32),
            pltpu.VMEM((CG,), jnp.int32), pltpu.VMEM((CG,), jnp.int32),
            pltpu.VMEM((CG,), jnp.int32), pltpu.VMEM((CG,), jnp.int32),
            pltpu.VMEM((CG, D), _f32), pltpu.VMEM((CG, D), _f32),
            pltpu.VMEM((CG, D), _f32), pltpu.VMEM((CG, D), _f32),
            pltpu.VMEM((CG, D), _f32),
            pltpu.VMEM((REMG,), jnp.int32), pltpu.VMEM((REMG,), jnp.int32),
            pltpu.VMEM((REMG, D), _f32), pltpu.VMEM((REMG, D), _f32),
            pltpu.SemaphoreType.DMA, pltpu.SemaphoreType.DMA,
            pltpu.SemaphoreType.DMA, pltpu.SemaphoreType.DMA,
            pltpu.SemaphoreType.DMA, pltpu.SemaphoreType.DMA,
        ],
    )
    def k(nf_h, src_h, dst_h, zc_h, gs_h, gd_h, cnt_h,
          csum_sh, sA, dA, sB, dB, srA, drA, srB, drB, ones,
          sidx2, didx2, srow2, drow2,
          semIA, semIB, semGA, semGB, semCA, semCB):
        c = lax.axis_index("c")
        t = lax.axis_index("s")
        base0 = (c * NS + t) * EPW

        pltpu.sync_copy(zc_h.at[pl.ds(t * RPT, RPT)],
                        csum_sh.at[pl.ds(t * RPT, RPT)])

        def fill(i, carry):
            ones[i // 8, pl.ds((i % 8) * 16, 16)] = jnp.ones((16,), _f32)
            return carry
        lax.fori_loop(0, CG * D // 16, fill, 0)

        plsc.subcore_barrier()

        pltpu.async_copy(src_h.at[pl.ds(base0, CG)], sA, semIA)
        pltpu.async_copy(dst_h.at[pl.ds(base0, CG)], dA, semIA)

        npair = NCG // 2

        def pair(m, carry):
            baseA = base0 + (2 * m) * CG
            baseB = baseA + CG
            baseN = jnp.where(m == npair - 1,
                              base0, base0 + (2 * m + 2) * CG)
            pltpu.make_async_copy(src_h.at[pl.ds(baseA, CG)], sA, semIA).wait()
            pltpu.make_async_copy(dst_h.at[pl.ds(baseA, CG)], dA, semIA).wait()
            cpA1 = pltpu.async_copy(nf_h.at[sA], srA, semGA)
            cpA2 = pltpu.async_copy(nf_h.at[dA], drA, semGA)
            pltpu.async_copy(src_h.at[pl.ds(baseB, CG)], sB, semIB)
            pltpu.async_copy(dst_h.at[pl.ds(baseB, CG)], dB, semIB)
            ccA = pltpu.async_copy(ones, csum_sh.at[dA], semCA, add=True)
            pltpu.make_async_copy(src_h.at[pl.ds(baseB, CG)], sB, semIB).wait()
            pltpu.make_async_copy(dst_h.at[pl.ds(baseB, CG)], dB, semIB).wait()
            cpB1 = pltpu.async_copy(nf_h.at[sB], srB, semGB)
            cpB2 = pltpu.async_copy(nf_h.at[dB], drB, semGB)
            ccB = pltpu.async_copy(ones, csum_sh.at[dB], semCB, add=True)
            cpA1.wait()
            pltpu.sync_copy(srA, gs_h.at[pl.ds(baseA, CG)])
            cpA2.wait()
            pltpu.sync_copy(drA, gd_h.at[pl.ds(baseA, CG)])
            cpB1.wait()
            pltpu.sync_copy(srB, gs_h.at[pl.ds(baseB, CG)])
            cpB2.wait()
            pltpu.sync_copy(drB, gd_h.at[pl.ds(baseB, CG)])
            ccA.wait()
            ccB.wait()
            pltpu.async_copy(src_h.at[pl.ds(baseN, CG)], sA, semIA)
            pltpu.async_copy(dst_h.at[pl.ds(baseN, CG)], dA, semIA)
            return carry

        lax.fori_loop(0, npair, pair, 0)

        pltpu.make_async_copy(src_h.at[pl.ds(base0, CG)], sA, semIA).wait()
        pltpu.make_async_copy(dst_h.at[pl.ds(base0, CG)], dA, semIA).wait()

        base = base0 + NCG * CG
        pltpu.sync_copy(src_h.at[pl.ds(base, REMG)], sidx2)
        pltpu.sync_copy(dst_h.at[pl.ds(base, REMG)], didx2)
        cp1 = pltpu.async_copy(nf_h.at[sidx2], srow2, semGA)
        cp2 = pltpu.async_copy(nf_h.at[didx2], drow2, semGB)
        pltpu.sync_copy(ones.at[0:REMG], csum_sh.at[didx2], add=True)
        cp1.wait()
        pltpu.sync_copy(srow2, gs_h.at[pl.ds(base, REMG)])
        cp2.wait()
        pltpu.sync_copy(drow2, gd_h.at[pl.ds(base, REMG)])

        plsc.subcore_barrier()
        pltpu.sync_copy(csum_sh.at[pl.ds(t * RPT, RPT)],
                        cnt_h.at[c, pl.ds(t * RPT, RPT)])

    return k(nf, src, dst, zc)


def _sc_scatter(hep, dst, zn):
    mesh = plsc.VectorSubcoreMesh(core_axis_name="c", subcore_axis_name="s")

    @functools.partial(
        pl.kernel,
        out_type=jax.ShapeDtypeStruct((NC, NPAD, D), _f32),
        mesh=mesh,
        scratch_types=[
            pltpu.VMEM_SHARED((NPAD, D), _f32),
            pltpu.VMEM((CH,), jnp.int32), pltpu.VMEM((CH,), jnp.int32),
            pltpu.VMEM((CH, D), _f32), pltpu.VMEM((CH, D), _f32),
            pltpu.VMEM((REM,), jnp.int32),
            pltpu.VMEM((REM, D), _f32),
            pltpu.SemaphoreType.DMA, pltpu.SemaphoreType.DMA,
            pltpu.SemaphoreType.DMA, pltpu.SemaphoreType.DMA,
        ],
    )
    def k(hep_h, dst_h, zn_h, parts_h, hsum_sh,
          dA, dB, rowsA, rowsB, didx2, rows2,
          semIA, semIB, semSA, semSB):
        c = lax.axis_index("c")
        t = lax.axis_index("s")

        pltpu.sync_copy(zn_h.at[pl.ds(t * RPT, RPT)],
                        hsum_sh.at[pl.ds(t * RPT, RPT)])
        plsc.subcore_barrier()

        base0 = (c * NS + t) * EPW
        npair = NCH // 2

        pltpu.async_copy(dst_h.at[pl.ds(base0, CH)], dA, semIA)
        pltpu.async_copy(hep_h.at[pl.ds(base0, CH)], rowsA, semIA)

        def pair(m, carry):
            baseA = base0 + (2 * m) * CH
            baseB = baseA + CH
            baseN = jnp.where(m == npair - 1,
                              base0, base0 + (2 * m + 2) * CH)
            pltpu.make_async_copy(dst_h.at[pl.ds(baseA, CH)], dA, semIA).wait()
            pltpu.make_async_copy(hep_h.at[pl.ds(baseA, CH)], rowsA, semIA).wait()
            scA = pltpu.async_copy(rowsA, hsum_sh.at[dA], semSA, add=True)
            pltpu.async_copy(dst_h.at[pl.ds(baseB, CH)], dB, semIB)
            pltpu.async_copy(hep_h.at[pl.ds(baseB, CH)], rowsB, semIB)
            pltpu.make_async_copy(dst_h.at[pl.ds(baseB, CH)], dB, semIB).wait()
            pltpu.make_async_copy(hep_h.at[pl.ds(baseB, CH)], rowsB, semIB).wait()
            scB = pltpu.async_copy(rowsB, hsum_sh.at[dB], semSB, add=True)
            scA.wait()
            scB.wait()
            pltpu.async_copy(dst_h.at[pl.ds(baseN, CH)], dA, semIA)
            pltpu.async_copy(hep_h.at[pl.ds(baseN, CH)], rowsA, semIA)
            return carry

        lax.fori_loop(0, npair, pair, 0)

        pltpu.make_async_copy(dst_h.at[pl.ds(base0, CH)], dA, semIA).wait()
        pltpu.make_async_copy(hep_h.at[pl.ds(base0, CH)], rowsA, semIA).wait()

        base = base0 + NCH * CH
        pltpu.sync_copy(dst_h.at[pl.ds(base, REM)], didx2)
        pltpu.sync_copy(hep_h.at[pl.ds(base, REM)], rows2)
        pltpu.sync_copy(rows2, hsum_sh.at[didx2], add=True)

        plsc.subcore_barrier()

        pltpu.sync_copy(hsum_sh.at[pl.ds(t * RPT, RPT)],
                        parts_h.at[c, pl.ds(t * RPT, RPT)])

    return k(hep, dst, zn)


def _edge1(gs, gd, ef, gf3, w1t, b1):
    def body(gs_r, gd_r, ef_r, gf_r, w_r, b_r, h1_r, st_r):
        i = pl.program_id(0)
        gfb = gf_r[0].astype(_bf16)
        x = jnp.concatenate(
            [gs_r[...].astype(_bf16), gd_r[...].astype(_bf16),
             ef_r[...].astype(_bf16),
             jnp.broadcast_to(gfb, (TE, D))], axis=1)
        h = jnp.dot(x, w_r[...], preferred_element_type=_f32) + b_r[...]
        h1_r[...] = h.astype(jnp.bfloat16)
        s = jnp.sum(h, axis=0, keepdims=True)
        q = jnp.sum(h * h, axis=0, keepdims=True)
        st = jnp.concatenate([s, q], axis=0)

        @pl.when(i == 0)
        def _():
            st_r[...] = jnp.zeros_like(st_r)
        st_r[...] += st

    return pl.pallas_call(
        body,
        grid=(GE,),
        in_specs=[
            pl.BlockSpec((TE, D), lambda i: (i, 0)),
            pl.BlockSpec((TE, D), lambda i: (i, 0)),
            pl.BlockSpec((TE, D), lambda i: (i, 0)),
            pl.BlockSpec((1, 1, D), lambda i: (i // TPG, 0, 0)),
            pl.BlockSpec((4 * D, 4 * D), lambda i: (0, 0)),
            pl.BlockSpec((1, 4 * D), lambda i: (0, 0)),
        ],
        out_specs=[
            pl.BlockSpec((TE, 4 * D), lambda i: (i, 0)),
            pl.BlockSpec((2, 4 * D), lambda i: (0, 0)),
        ],
        out_shape=[jax.ShapeDtypeStruct((E, 4 * D), jnp.bfloat16),
                   jax.ShapeDtypeStruct((2, 4 * D), _f32)],
    )(gs, gd, ef, gf3, w1t, b1)


def _edge2(h1, st1, g1, be1, w2t, b2):
    def body(h1_r, st_r, g_r, be_r, w_r, b_r, h2_r, st2_r):
        i = pl.program_id(0)
        mu = st_r[0:1, :] * (1.0 / E)
        var = st_r[1:2, :] * (1.0 / E) - mu * mu
        sc = g_r[...] * lax.rsqrt(var + EPS)
        sh = be_r[...] - sc * mu
        a = _softplus(h1_r[...].astype(_f32) * sc + sh)
        h2 = jnp.dot(a.astype(_bf16), w_r[...],
                     preferred_element_type=_f32) + b_r[...]
        h2_r[...] = h2.astype(_bf16)
        s = jnp.sum(h2, axis=0, keepdims=True)
        q = jnp.sum(h2 * h2, axis=0, keepdims=True)
        st = jnp.concatenate([s, q], axis=0)

        @pl.when(i == 0)
        def _():
            st2_r[...] = jnp.zeros_like(st2_r)
        st2_r[...] += st

    return pl.pallas_call(
        body,
        grid=(GE,),
        in_specs=[
            pl.BlockSpec((TE, 4 * D), lambda i: (i, 0)),
            pl.BlockSpec((2, 4 * D), lambda i: (0, 0)),
            pl.BlockSpec((1, 4 * D), lambda i: (0, 0)),
            pl.BlockSpec((1, 4 * D), lambda i: (0, 0)),
            pl.BlockSpec((4 * D, D), lambda i: (0, 0)),
            pl.BlockSpec((1, D), lambda i: (0, 0)),
        ],
        out_specs=[
            pl.BlockSpec((TE, D), lambda i: (i, 0)),
            pl.BlockSpec((2, D), lambda i: (0, 0)),
        ],
        out_shape=[jax.ShapeDtypeStruct((E, D), jnp.bfloat16),
                   jax.ShapeDtypeStruct((2, D), _f32)],
    )(h1, st1, g1, be1, w2t, b2)


def _edge3(h2, st2, g2, be2):
    def body(h2_r, st_r, g_r, be_r, he_r):
        mu = st_r[0:1, :] * (1.0 / E)
        var = st_r[1:2, :] * (1.0 / E) - mu * mu
        sc = g_r[...] * lax.rsqrt(var + EPS)
        sh = be_r[...] - sc * mu
        he_r[...] = _softplus(h2_r[...].astype(_f32) * sc + sh)

    return pl.pallas_call(
        body,
        grid=(GE,),
        in_specs=[
            pl.BlockSpec((TE, D), lambda i: (i, 0)),
            pl.BlockSpec((2, D), lambda i: (0, 0)),
            pl.BlockSpec((1, D), lambda i: (0, 0)),
            pl.BlockSpec((1, D), lambda i: (0, 0)),
        ],
        out_specs=pl.BlockSpec((TE, D), lambda i: (i, 0)),
        out_shape=jax.ShapeDtypeStruct((E, D), _f32),
    )(h2, st2, g2, be2)


def _node1(nf, parts, cparts, gf3, w3t, b3):
    def body(nf_r, p_r, c_r, gf_r, w_r, b_r, h3_r, st_r, ge_r):
        b = pl.program_id(0)
        pv = p_r[...]
        hs = pv[0] + pv[1]
        cv = c_r[...]
        cnt = (cv[0] + cv[1])[:, 0:1]
        have = hs / jnp.maximum(cnt, 1.0)
        gfb = gf_r[0]
        x = jnp.concatenate(
            [nf_r[...], have, jnp.broadcast_to(gfb, (NP, D))], axis=1)
        h = jnp.dot(x, w_r[...], preferred_element_type=_f32) + b_r[...]
        h3_r[...] = h
        s = jnp.sum(h, axis=0, keepdims=True)
        q = jnp.sum(h * h, axis=0, keepdims=True)
        st = jnp.concatenate([s, q], axis=0)

        @pl.when(b == 0)
        def _():
            st_r[...] = jnp.zeros_like(st_r)
        st_r[...] += st
        ge_r[...] = (jnp.sum(have, axis=0, keepdims=True)
                     * (1.0 / NP))[None]

    return pl.pallas_call(
        body,
        grid=(B,),
        in_specs=[
            pl.BlockSpec((NP, D), lambda b: (b, 0)),
            pl.BlockSpec((NC, NP, D), lambda b: (0, b, 0)),
            pl.BlockSpec((NC, NP, D), lambda b: (0, b, 0)),
            pl.BlockSpec((1, 1, D), lambda b: (b, 0, 0)),
            pl.BlockSpec((3 * D, 3 * D), lambda b: (0, 0)),
            pl.BlockSpec((1, 3 * D), lambda b: (0, 0)),
        ],
        out_specs=[
            pl.BlockSpec((NP, 3 * D), lambda b: (b, 0)),
            pl.BlockSpec((2, 3 * D), lambda b: (0, 0)),
            pl.BlockSpec((1, 1, D), lambda b: (b, 0, 0)),
        ],
        out_shape=[jax.ShapeDtypeStruct((N, 3 * D), _f32),
                   jax.ShapeDtypeStruct((2, 3 * D), _f32),
                   jax.ShapeDtypeStruct((B, 1, D), _f32)],
    )(nf, parts, cparts, gf3, w3t, b3)


def _node2(h3, st3, g3, be3, w4t, b4):
    def body(h3_r, st_r, g_r, be_r, w_r, b_r, h4_r, st4_r):
        b = pl.program_id(0)
        mu = st_r[0:1, :] * (1.0 / N)
        var = st_r[1:2, :] * (1.0 / N) - mu * mu
        sc = g_r[...] * lax.rsqrt(var + EPS)
        sh = be_r[...] - sc * mu
        a = _softplus(h3_r[...] * sc + sh)
        h4 = jnp.dot(a, w_r[...], preferred_element_type=_f32) + b_r[...]
        h4_r[...] = h4
        s = jnp.sum(h4, axis=0, keepdims=True)
        q = jnp.sum(h4 * h4, axis=0, keepdims=True)
        st = jnp.concatenate([s, q], axis=0)

        @pl.when(b == 0)
        def _():
            st4_r[...] = jnp.zeros_like(st4_r)
        st4_r[...] += st

    return pl.pallas_call(
        body,
        grid=(B,),
        in_specs=[
            pl.BlockSpec((NP, 3 * D), lambda b: (b, 0)),
            pl.BlockSpec((2, 3 * D), lambda b: (0, 0)),
            pl.BlockSpec((1, 3 * D), lambda b: (0, 0)),
            pl.BlockSpec((1, 3 * D), lambda b: (0, 0)),
            pl.BlockSpec((3 * D, D), lambda b: (0, 0)),
            pl.BlockSpec((1, D), lambda b: (0, 0)),
        ],
        out_specs=[
            pl.BlockSpec((NP, D), lambda b: (b, 0)),
            pl.BlockSpec((2, D), lambda b: (0, 0)),
        ],
        out_shape=[jax.ShapeDtypeStruct((N, D), _f32),
                   jax.ShapeDtypeStruct((2, D), _f32)],
    )(h3, st3, g3, be3, w4t, b4)


def _node3(h4, st4, g4, be4):
    def body(h4_r, st_r, g_r, be_r, hn_r, gn_r):
        mu = st_r[0:1, :] * (1.0 / N)
        var = st_r[1:2, :] * (1.0 / N) - mu * mu
        sc = g_r[...] * lax.rsqrt(var + EPS)
        sh = be_r[...] - sc * mu
        hn = _softplus(h4_r[...] * sc + sh)
        hn_r[...] = hn
        gn_r[...] = (jnp.sum(hn, axis=0, keepdims=True) * (1.0 / NP))[None]

    return pl.pallas_call(
        body,
        grid=(B,),
        in_specs=[
            pl.BlockSpec((NP, D), lambda b: (b, 0)),
            pl.BlockSpec((2, D), lambda b: (0, 0)),
            pl.BlockSpec((1, D), lambda b: (0, 0)),
            pl.BlockSpec((1, D), lambda b: (0, 0)),
        ],
        out_specs=[
            pl.BlockSpec((NP, D), lambda b: (b, 0)),
            pl.BlockSpec((1, 1, D), lambda b: (b, 0, 0)),
        ],
        out_shape=[jax.ShapeDtypeStruct((N, D), _f32),
                   jax.ShapeDtypeStruct((B, 1, D), _f32)],
    )(h4, st4, g4, be4)


def _glob(gn, ge, gf, wg1t, bg1, gg1, beg1, wg2t, bg2, gg2, beg2):
    def bn(h, g, be):
        mu = jnp.mean(h, axis=0, keepdims=True)
        var = jnp.mean(h * h, axis=0, keepdims=True) - mu * mu
        return _softplus(g * ((h - mu) * lax.rsqrt(var + EPS)) + be)

    def body(gn_r, ge_r, gf_r, w1_r, b1_r, g1_r, be1_r,
             w2_r, b2_r, g2_r, be2_r, hg_r):
        x = jnp.concatenate([gn_r[...], ge_r[...], gf_r[...]], axis=1)
        h = jnp.dot(x, w1_r[...], preferred_element_type=_f32) + b1_r[...]
        h = bn(h, g1_r[...], be1_r[...])
        h2 = jnp.dot(h, w2_r[...], preferred_element_type=_f32) + b2_r[...]
        hg_r[...] = bn(h2, g2_r[...], be2_r[...])

    return pl.pallas_call(
        body,
        out_shape=jax.ShapeDtypeStruct((B, D), _f32),
    )(gn, ge, gf, wg1t, bg1, gg1, beg1, wg2t, bg2, gg2, beg2)


def kernel(node_feats, edge_feats, global_feats, params, src, dst,
           node_gid, batch_num_nodes, batch_num_edges):
    p = params
    r1 = lambda v: v.reshape(1, -1)
    gf3 = global_feats.reshape(B, 1, D)

    gs, gd, cparts = _sc_gather(node_feats, src, dst,
                                jnp.zeros((NPAD, D), _f32))
    h1, st1 = _edge1(gs, gd, edge_feats, gf3,
                     p['e1_W'].T.astype(_bf16), r1(p['e1_b']))
    h2, st2 = _edge2(h1, st1, r1(p['e1_g']), r1(p['e1_be']),
                     p['e2_W'].T.astype(_bf16), r1(p['e2_b']))
    he = _edge3(h2, st2, r1(p['e2_g']), r1(p['e2_be']))

    parts = _sc_scatter(he, dst, jnp.zeros((NPAD, D), _f32))

    h3, st3, gedge = _node1(node_feats, parts, cparts, gf3,
                            p['n1_W'].T, r1(p['n1_b']))
    h4, st4 = _node2(h3, st3, r1(p['n1_g']), r1(p['n1_be']),
                     p['n2_W'].T, r1(p['n2_b']))
    hn, gnode = _node3(h4, st4, r1(p['n2_g']), r1(p['n2_be']))

    hg = _glob(gnode.reshape(B, D), gedge.reshape(B, D), global_feats,
               p['g1_W'].T, r1(p['g1_b']), r1(p['g1_g']), r1(p['g1_be']),
               p['g2_W'].T, r1(p['g2_b']), r1(p['g2_g']), r1(p['g2_be']))

    return hn, he, hg


# speedup vs baseline: 1.2012x; 1.0028x over previous
import functools

import numpy as np

import jax
import jax.numpy as jnp
from jax import lax
from jax.experimental import pallas as pl
from jax.experimental.pallas import tpu as pltpu
from jax.experimental.pallas import tpu_sc as plsc

D = 128
B = 10
NP = 1000
EP = 32000
N = B * NP
E = B * EP
EPS = 1e-5

NC, NS = 2, 16
NW = NC * NS
EPW = E // NW
CH = 128
NCH = EPW // CH
REM = EPW - NCH * CH
CG = 64
NCG = EPW // CG
REMG = EPW - NCG * CG
NPAD = 10112
RPT = NPAD // NS

TE = 2000
GE = E // TE
TPG = EP // TE

_f32 = jnp.float32
_bf16 = jnp.bfloat16


def _softplus(x):
    return jnp.maximum(x, 0.0) + jnp.log(1.0 + jnp.exp(-jnp.abs(x)))


def _sc_gather(nf, src, dst, zc):
    mesh = plsc.VectorSubcoreMesh(core_axis_name="c", subcore_axis_name="s")

    @functools.partial(
        pl.kernel,
        out_type=(jax.ShapeDtypeStruct((E, D), _f32),
                  jax.ShapeDtypeStruct((E, D), _f32),
                  jax.ShapeDtypeStruct((NC, NPAD, D), _f32)),
        mesh=mesh,
        scratch_types=[
            pltpu.VMEM_SHARED((NPAD, D), _f32),
            pltpu.VMEM((CG,), jnp.int32), pltpu.VMEM((CG,), jnp.int32),
            pltpu.VMEM((CG,), jnp.int32), pltpu.VMEM((CG,), jnp.int32),
            pltpu.VMEM((CG, D), _f32), pltpu.VMEM((CG, D), _f32),
            pltpu.VMEM((CG, D), _f32), pltpu.VMEM((CG, D), _f32),
            pltpu.VMEM((CG, D), _f32),
            pltpu.VMEM((REMG,), jnp.int32), pltpu.VMEM((REMG,), jnp.int32),
            pltpu.VMEM((REMG, D), _f32), pltpu.VMEM((REMG, D), _f32),
            pltpu.SemaphoreType.DMA, pltpu.SemaphoreType.DMA,
            pltpu.SemaphoreType.DMA, pltpu.SemaphoreType.DMA,
            pltpu.SemaphoreType.DMA, pltpu.SemaphoreType.DMA,
        ],
    )
    def k(nf_h, src_h, dst_h, zc_h, gs_h, gd_h, cnt_h,
          csum_sh, sA, dA, sB, dB, srA, drA, srB, drB, ones,
          sidx2, didx2, srow2, drow2,
          semIA, semIB, semGA, semGB, semCA, semCB):
        c = lax.axis_index("c")
        t = lax.axis_index("s")
        base0 = (c * NS + t) * EPW

        pltpu.sync_copy(zc_h.at[pl.ds(t * RPT, RPT)],
                        csum_sh.at[pl.ds(t * RPT, RPT)])

        def fill(i, carry):
            ones[i // 8, pl.ds((i % 8) * 16, 16)] = jnp.ones((16,), _f32)
            return carry
        lax.fori_loop(0, CG * D // 16, fill, 0)

        plsc.subcore_barrier()

        pltpu.async_copy(src_h.at[pl.ds(base0, CG)], sA, semIA)
        pltpu.async_copy(dst_h.at[pl.ds(base0, CG)], dA, semIA)

        npair = NCG // 2

        def pair(m, carry):
            baseA = base0 + (2 * m) * CG
            baseB = baseA + CG
            baseN = jnp.where(m == npair - 1,
                              base0, base0 + (2 * m + 2) * CG)
            pltpu.make_async_copy(src_h.at[pl.ds(baseA, CG)], sA, semIA).wait()
            pltpu.make_async_copy(dst_h.at[pl.ds(baseA, CG)], dA, semIA).wait()
            cpA1 = pltpu.async_copy(nf_h.at[sA], srA, semGA)
            cpA2 = pltpu.async_copy(nf_h.at[dA], drA, semGA)
            pltpu.async_copy(src_h.at[pl.ds(baseB, CG)], sB, semIB)
            pltpu.async_copy(dst_h.at[pl.ds(baseB, CG)], dB, semIB)
            ccA = pltpu.async_copy(ones, csum_sh.at[dA], semCA, add=True)
            pltpu.make_async_copy(src_h.at[pl.ds(baseB, CG)], sB, semIB).wait()
            pltpu.make_async_copy(dst_h.at[pl.ds(baseB, CG)], dB, semIB).wait()
            cpB1 = pltpu.async_copy(nf_h.at[sB], srB, semGB)
            cpB2 = pltpu.async_copy(nf_h.at[dB], drB, semGB)
            ccB = pltpu.async_copy(ones, csum_sh.at[dB], semCB, add=True)
            cpA1.wait()
            pltpu.sync_copy(srA, gs_h.at[pl.ds(baseA, CG)])
            cpA2.wait()
            pltpu.sync_copy(drA, gd_h.at[pl.ds(baseA, CG)])
            cpB1.wait()
            pltpu.sync_copy(srB, gs_h.at[pl.ds(baseB, CG)])
            cpB2.wait()
            pltpu.sync_copy(drB, gd_h.at[pl.ds(baseB, CG)])
            ccA.wait()
            ccB.wait()
            pltpu.async_copy(src_h.at[pl.ds(baseN, CG)], sA, semIA)
            pltpu.async_copy(dst_h.at[pl.ds(baseN, CG)], dA, semIA)
            return carry

        lax.fori_loop(0, npair, pair, 0)

        pltpu.make_async_copy(src_h.at[pl.ds(base0, CG)], sA, semIA).wait()
        pltpu.make_async_copy(dst_h.at[pl.ds(base0, CG)], dA, semIA).wait()

        base = base0 + NCG * CG
        pltpu.sync_copy(src_h.at[pl.ds(base, REMG)], sidx2)
        pltpu.sync_copy(dst_h.at[pl.ds(base, REMG)], didx2)
        cp1 = pltpu.async_copy(nf_h.at[sidx2], srow2, semGA)
        cp2 = pltpu.async_copy(nf_h.at[didx2], drow2, semGB)
        pltpu.sync_copy(ones.at[0:REMG], csum_sh.at[didx2], add=True)
        cp1.wait()
        pltpu.sync_copy(srow2, gs_h.at[pl.ds(base, REMG)])
        cp2.wait()
        pltpu.sync_copy(drow2, gd_h.at[pl.ds(base, REMG)])

        plsc.subcore_barrier()
        pltpu.sync_copy(csum_sh.at[pl.ds(t * RPT, RPT)],
                        cnt_h.at[c, pl.ds(t * RPT, RPT)])

    return k(nf, src, dst, zc)


def _sc_scatter(hep, dst, zn):
    mesh = plsc.VectorSubcoreMesh(core_axis_name="c", subcore_axis_name="s")

    @functools.partial(
        pl.kernel,
        out_type=jax.ShapeDtypeStruct((NC, NPAD, D), _f32),
        mesh=mesh,
        scratch_types=[
            pltpu.VMEM_SHARED((NPAD, D), _f32),
            pltpu.VMEM((CH,), jnp.int32), pltpu.VMEM((CH,), jnp.int32),
            pltpu.VMEM((CH, D), _f32), pltpu.VMEM((CH, D), _f32),
            pltpu.VMEM((REM,), jnp.int32),
            pltpu.VMEM((REM, D), _f32),
            pltpu.SemaphoreType.DMA, pltpu.SemaphoreType.DMA,
            pltpu.SemaphoreType.DMA, pltpu.SemaphoreType.DMA,
        ],
    )
    def k(hep_h, dst_h, zn_h, parts_h, hsum_sh,
          dA, dB, rowsA, rowsB, didx2, rows2,
          semIA, semIB, semSA, semSB):
        c = lax.axis_index("c")
        t = lax.axis_index("s")

        pltpu.sync_copy(zn_h.at[pl.ds(t * RPT, RPT)],
                        hsum_sh.at[pl.ds(t * RPT, RPT)])
        plsc.subcore_barrier()

        base0 = (c * NS + t) * EPW
        npair = NCH // 2

        pltpu.async_copy(dst_h.at[pl.ds(base0, CH)], dA, semIA)
        pltpu.async_copy(hep_h.at[pl.ds(base0, CH)], rowsA, semIA)

        def pair(m, carry):
            baseA = base0 + (2 * m) * CH
            baseB = baseA + CH
            baseN = jnp.where(m == npair - 1,
                              base0, base0 + (2 * m + 2) * CH)
            pltpu.make_async_copy(dst_h.at[pl.ds(baseA, CH)], dA, semIA).wait()
            pltpu.make_async_copy(hep_h.at[pl.ds(baseA, CH)], rowsA, semIA).wait()
            scA = pltpu.async_copy(rowsA, hsum_sh.at[dA], semSA, add=True)
            pltpu.async_copy(dst_h.at[pl.ds(baseB, CH)], dB, semIB)
            pltpu.async_copy(hep_h.at[pl.ds(baseB, CH)], rowsB, semIB)
            pltpu.make_async_copy(dst_h.at[pl.ds(baseB, CH)], dB, semIB).wait()
            pltpu.make_async_copy(hep_h.at[pl.ds(baseB, CH)], rowsB, semIB).wait()
            scB = pltpu.async_copy(rowsB, hsum_sh.at[dB], semSB, add=True)
            scA.wait()
            scB.wait()
            pltpu.async_copy(dst_h.at[pl.ds(baseN, CH)], dA, semIA)
            pltpu.async_copy(hep_h.at[pl.ds(baseN, CH)], rowsA, semIA)
            return carry

        lax.fori_loop(0, npair, pair, 0)

        pltpu.make_async_copy(dst_h.at[pl.ds(base0, CH)], dA, semIA).wait()
        pltpu.make_async_copy(hep_h.at[pl.ds(base0, CH)], rowsA, semIA).wait()

        base = base0 + NCH * CH
        pltpu.sync_copy(dst_h.at[pl.ds(base, REM)], didx2)
        pltpu.sync_copy(hep_h.at[pl.ds(base, REM)], rows2)
        pltpu.sync_copy(rows2, hsum_sh.at[didx2], add=True)

        plsc.subcore_barrier()

        pltpu.sync_copy(hsum_sh.at[pl.ds(t * RPT, RPT)],
                        parts_h.at[c, pl.ds(t * RPT, RPT)])

    return k(hep, dst, zn)


def _edge1(gs, gd, ef, gf3, w1t, b1):
    def body(gs_r, gd_r, ef_r, gf_r, w_r, b_r, h1_r, st_r):
        i = pl.program_id(0)
        gfb = gf_r[0].astype(_bf16)
        x = jnp.concatenate(
            [gs_r[...].astype(_bf16), gd_r[...].astype(_bf16),
             ef_r[...].astype(_bf16),
             jnp.broadcast_to(gfb, (TE, D))], axis=1)
        h = jnp.dot(x, w_r[...], preferred_element_type=_f32) + b_r[...]
        h1_r[...] = h.astype(jnp.bfloat16)
        s = jnp.sum(h, axis=0, keepdims=True)
        q = jnp.sum(h * h, axis=0, keepdims=True)
        st = jnp.concatenate([s, q], axis=0)

        @pl.when(i == 0)
        def _():
            st_r[...] = jnp.zeros_like(st_r)
        st_r[...] += st

    return pl.pallas_call(
        body,
        grid=(GE,),
        in_specs=[
            pl.BlockSpec((TE, D), lambda i: (i, 0)),
            pl.BlockSpec((TE, D), lambda i: (i, 0)),
            pl.BlockSpec((TE, D), lambda i: (i, 0)),
            pl.BlockSpec((1, 1, D), lambda i: (i // TPG, 0, 0)),
            pl.BlockSpec((4 * D, 4 * D), lambda i: (0, 0)),
            pl.BlockSpec((1, 4 * D), lambda i: (0, 0)),
        ],
        out_specs=[
            pl.BlockSpec((TE, 4 * D), lambda i: (i, 0)),
            pl.BlockSpec((2, 4 * D), lambda i: (0, 0)),
        ],
        out_shape=[jax.ShapeDtypeStruct((E, 4 * D), jnp.bfloat16),
                   jax.ShapeDtypeStruct((2, 4 * D), _f32)],
    )(gs, gd, ef, gf3, w1t, b1)


def _edge2(h1, st1, g1, be1, w2t, b2):
    def body(h1_r, st_r, g_r, be_r, w_r, b_r, h2_r, st2_r):
        i = pl.program_id(0)
        mu = st_r[0:1, :] * (1.0 / E)
        var = st_r[1:2, :] * (1.0 / E) - mu * mu
        sc = g_r[...] * lax.rsqrt(var + EPS)
        sh = be_r[...] - sc * mu
        a = _softplus(h1_r[...].astype(_f32) * sc + sh)
        h2 = jnp.dot(a.astype(_bf16), w_r[...],
                     preferred_element_type=_f32) + b_r[...]
        h2_r[...] = h2.astype(_bf16)
        s = jnp.sum(h2, axis=0, keepdims=True)
        q = jnp.sum(h2 * h2, axis=0, keepdims=True)
        st = jnp.concatenate([s, q], axis=0)

        @pl.when(i == 0)
        def _():
            st2_r[...] = jnp.zeros_like(st2_r)
        st2_r[...] += st

    return pl.pallas_call(
        body,
        grid=(GE,),
        in_specs=[
            pl.BlockSpec((TE, 4 * D), lambda i: (i, 0)),
            pl.BlockSpec((2, 4 * D), lambda i: (0, 0)),
            pl.BlockSpec((1, 4 * D), lambda i: (0, 0)),
            pl.BlockSpec((1, 4 * D), lambda i: (0, 0)),
            pl.BlockSpec((4 * D, D), lambda i: (0, 0)),
            pl.BlockSpec((1, D), lambda i: (0, 0)),
        ],
        out_specs=[
            pl.BlockSpec((TE, D), lambda i: (i, 0)),
            pl.BlockSpec((2, D), lambda i: (0, 0)),
        ],
        out_shape=[jax.ShapeDtypeStruct((E, D), jnp.bfloat16),
                   jax.ShapeDtypeStruct((2, D), _f32)],
    )(h1, st1, g1, be1, w2t, b2)


def _edge3(h2, st2, g2, be2):
    def body(h2_r, st_r, g_r, be_r, he_r):
        mu = st_r[0:1, :] * (1.0 / E)
        var = st_r[1:2, :] * (1.0 / E) - mu * mu
        sc = g_r[...] * lax.rsqrt(var + EPS)
        sh = be_r[...] - sc * mu
        he_r[...] = _softplus(h2_r[...].astype(_f32) * sc + sh)

    return pl.pallas_call(
        body,
        grid=(GE,),
        in_specs=[
            pl.BlockSpec((TE, D), lambda i: (i, 0)),
            pl.BlockSpec((2, D), lambda i: (0, 0)),
            pl.BlockSpec((1, D), lambda i: (0, 0)),
            pl.BlockSpec((1, D), lambda i: (0, 0)),
        ],
        out_specs=pl.BlockSpec((TE, D), lambda i: (i, 0)),
        out_shape=jax.ShapeDtypeStruct((E, D), _f32),
    )(h2, st2, g2, be2)


def _node1(nf, parts, cparts, gf3, w3t, b3):
    def body(nf_r, p_r, c_r, gf_r, w_r, b_r, h3_r, st_r, ge_r):
        b = pl.program_id(0)
        pv = p_r[...]
        hs = pv[0] + pv[1]
        cv = c_r[...]
        cnt = (cv[0] + cv[1])[:, 0:1]
        have = hs / jnp.maximum(cnt, 1.0)
        gfb = gf_r[0]
        x = jnp.concatenate(
            [nf_r[...], have, jnp.broadcast_to(gfb, (NP, D))], axis=1)
        h = jnp.dot(x, w_r[...], preferred_element_type=_f32) + b_r[...]
        h3_r[...] = h
        s = jnp.sum(h, axis=0, keepdims=True)
        q = jnp.sum(h * h, axis=0, keepdims=True)
        st = jnp.concatenate([s, q], axis=0)

        @pl.when(b == 0)
        def _():
            st_r[...] = jnp.zeros_like(st_r)
        st_r[...] += st
        ge_r[...] = (jnp.sum(have, axis=0, keepdims=True)
                     * (1.0 / NP))[None]

    return pl.pallas_call(
        body,
        grid=(B,),
        in_specs=[
            pl.BlockSpec((NP, D), lambda b: (b, 0)),
            pl.BlockSpec((NC, NP, D), lambda b: (0, b, 0)),
            pl.BlockSpec((NC, NP, D), lambda b: (0, b, 0)),
            pl.BlockSpec((1, 1, D), lambda b: (b, 0, 0)),
            pl.BlockSpec((3 * D, 3 * D), lambda b: (0, 0)),
            pl.BlockSpec((1, 3 * D), lambda b: (0, 0)),
        ],
        out_specs=[
            pl.BlockSpec((NP, 3 * D), lambda b: (b, 0)),
            pl.BlockSpec((2, 3 * D), lambda b: (0, 0)),
            pl.BlockSpec((1, 1, D), lambda b: (b, 0, 0)),
        ],
        out_shape=[jax.ShapeDtypeStruct((N, 3 * D), _f32),
                   jax.ShapeDtypeStruct((2, 3 * D), _f32),
                   jax.ShapeDtypeStruct((B, 1, D), _f32)],
    )(nf, parts, cparts, gf3, w3t, b3)


def _node2(h3, st3, g3, be3, w4t, b4):
    def body(h3_r, st_r, g_r, be_r, w_r, b_r, h4_r, st4_r):
        b = pl.program_id(0)
        mu = st_r[0:1, :] * (1.0 / N)
        var = st_r[1:2, :] * (1.0 / N) - mu * mu
        sc = g_r[...] * lax.rsqrt(var + EPS)
        sh = be_r[...] - sc * mu
        a = _softplus(h3_r[...] * sc + sh)
        h4 = jnp.dot(a, w_r[...], preferred_element_type=_f32) + b_r[...]
        h4_r[...] = h4
        s = jnp.sum(h4, axis=0, keepdims=True)
        q = jnp.sum(h4 * h4, axis=0, keepdims=True)
        st = jnp.concatenate([s, q], axis=0)

        @pl.when(b == 0)
        def _():
            st4_r[...] = jnp.zeros_like(st4_r)
        st4_r[...] += st

    return pl.pallas_call(
        body,
        grid=(B,),
        in_specs=[
            pl.BlockSpec((NP, 3 * D), lambda b: (b, 0)),
            pl.BlockSpec((2, 3 * D), lambda b: (0, 0)),
            pl.BlockSpec((1, 3 * D), lambda b: (0, 0)),
            pl.BlockSpec((1, 3 * D), lambda b: (0, 0)),
            pl.BlockSpec((3 * D, D), lambda b: (0, 0)),
            pl.BlockSpec((1, D), lambda b: (0, 0)),
        ],
        out_specs=[
            pl.BlockSpec((NP, D), lambda b: (b, 0)),
            pl.BlockSpec((2, D), lambda b: (0, 0)),
        ],
        out_shape=[jax.ShapeDtypeStruct((N, D), _f32),
                   jax.ShapeDtypeStruct((2, D), _f32)],
    )(h3, st3, g3, be3, w4t, b4)


def _node3(h4, st4, g4, be4):
    def body(h4_r, st_r, g_r, be_r, hn_r, gn_r):
        mu = st_r[0:1, :] * (1.0 / N)
        var = st_r[1:2, :] * (1.0 / N) - mu * mu
        sc = g_r[...] * lax.rsqrt(var + EPS)
        sh = be_r[...] - sc * mu
        hn = _softplus(h4_r[...] * sc + sh)
        hn_r[...] = hn
        gn_r[...] = (jnp.sum(hn, axis=0, keepdims=True) * (1.0 / NP))[None]

    return pl.pallas_call(
        body,
        grid=(B,),
        in_specs=[
            pl.BlockSpec((NP, D), lambda b: (b, 0)),
            pl.BlockSpec((2, D), lambda b: (0, 0)),
            pl.BlockSpec((1, D), lambda b: (0, 0)),
            pl.BlockSpec((1, D), lambda b: (0, 0)),
        ],
        out_specs=[
            pl.BlockSpec((NP, D), lambda b: (b, 0)),
            pl.BlockSpec((1, 1, D), lambda b: (b, 0, 0)),
        ],
        out_shape=[jax.ShapeDtypeStruct((N, D), _f32),
                   jax.ShapeDtypeStruct((B, 1, D), _f32)],
    )(h4, st4, g4, be4)


def _glob(gn, ge, gf, wg1t, bg1, gg1, beg1, wg2t, bg2, gg2, beg2):
    def bn(h, g, be):
        mu = jnp.mean(h, axis=0, keepdims=True)
        var = jnp.mean(h * h, axis=0, keepdims=True) - mu * mu
        return _softplus(g * ((h - mu) * lax.rsqrt(var + EPS)) + be)

    def body(gn_r, ge_r, gf_r, w1_r, b1_r, g1_r, be1_r,
             w2_r, b2_r, g2_r, be2_r, hg_r):
        x = jnp.concatenate([gn_r[...], ge_r[...], gf_r[...]], axis=1)
        h = jnp.dot(x, w1_r[...], preferred_element_type=_f32) + b1_r[...]
        h = bn(h, g1_r[...], be1_r[...])
        h2 = jnp.dot(h, w2_r[...], preferred_element_type=_f32) + b2_r[...]
        hg_r[...] = bn(h2, g2_r[...], be2_r[...])

    return pl.pallas_call(
        body,
        out_shape=jax.ShapeDtypeStruct((B, D), _f32),
    )(gn, ge, gf, wg1t, bg1, gg1, beg1, wg2t, bg2, gg2, beg2)


def _nodeglob(nf, parts, cnt, gf, w3t, b3, g3, be3, w4t, b4, g4, be4,
              wg1t, bg1, gg1, beg1, wg2t, bg2, gg2, beg2):
    def bn_rows(h, g, be, n):
        mu = jnp.mean(h, axis=0, keepdims=True)
        var = jnp.mean(h * h, axis=0, keepdims=True) - mu * mu
        return _softplus(g * ((h - mu) * lax.rsqrt(var + EPS)) + be)

    def body(nf_r, p_r, c_r, gf_r, w3_r, b3_r, g3_r, be3_r,
             w4_r, b4_r, g4_r, be4_r, wg1_r, bg1_r, gg1_r, beg1_r,
             wg2_r, bg2_r, gg2_r, beg2_r, hn_r, hg_r):
        pv = p_r[...]
        hs = pv[0] + pv[1]
        cv = c_r[...]
        cnt_ = (cv[0] + cv[1])
        have = hs / jnp.maximum(cnt_, 1.0)
        gfv = gf_r[...]
        gn = jnp.broadcast_to(gfv[:, None, :], (B, NP, D)).reshape(N, D)
        x = jnp.concatenate([nf_r[...], have, gn], axis=1)
        h3 = jnp.dot(x, w3_r[...], preferred_element_type=_f32) + b3_r[...]
        hn1 = bn_rows(h3, g3_r[...], be3_r[...], N)
        h4 = jnp.dot(hn1, w4_r[...], preferred_element_type=_f32) + b4_r[...]
        hn = bn_rows(h4, g4_r[...], be4_r[...], N)
        hn_r[...] = hn
        gnode = jnp.mean(hn.reshape(B, NP, D), axis=1)
        gedge = jnp.mean(have.reshape(B, NP, D), axis=1)
        xg = jnp.concatenate([gnode, gedge, gfv], axis=1)
        hg1 = jnp.dot(xg, wg1_r[...], preferred_element_type=_f32) + bg1_r[...]
        hg1 = bn_rows(hg1, gg1_r[...], beg1_r[...], B)
        hg2 = jnp.dot(hg1, wg2_r[...], preferred_element_type=_f32) + bg2_r[...]
        hg_r[...] = bn_rows(hg2, gg2_r[...], beg2_r[...], B)

    return pl.pallas_call(
        body,
        out_shape=[jax.ShapeDtypeStruct((N, D), _f32),
                   jax.ShapeDtypeStruct((B, D), _f32)],
    )(nf, parts, cnt, gf, w3t, b3, g3, be3, w4t, b4, g4, be4,
      wg1t, bg1, gg1, beg1, wg2t, bg2, gg2, beg2)


def kernel(node_feats, edge_feats, global_feats, params, src, dst,
           node_gid, batch_num_nodes, batch_num_edges):
    p = params
    r1 = lambda v: v.reshape(1, -1)
    gf3 = global_feats.reshape(B, 1, D)

    gs, gd, cparts = _sc_gather(node_feats, src, dst,
                                jnp.zeros((NPAD, D), _f32))
    h1, st1 = _edge1(gs, gd, edge_feats, gf3,
                     p['e1_W'].T.astype(_bf16), r1(p['e1_b']))
    h2, st2 = _edge2(h1, st1, r1(p['e1_g']), r1(p['e1_be']),
                     p['e2_W'].T.astype(_bf16), r1(p['e2_b']))
    he = _edge3(h2, st2, r1(p['e2_g']), r1(p['e2_be']))

    parts = _sc_scatter(he, dst, jnp.zeros((NPAD, D), _f32))

    hn, hg = _nodeglob(
        node_feats, parts[:, :N], cparts[:, :N, 0:1], global_feats,
        p['n1_W'].T, r1(p['n1_b']), r1(p['n1_g']), r1(p['n1_be']),
        p['n2_W'].T, r1(p['n2_b']), r1(p['n2_g']), r1(p['n2_be']),
        p['g1_W'].T, r1(p['g1_b']), r1(p['g1_g']), r1(p['g1_be']),
        p['g2_W'].T, r1(p['g2_b']), r1(p['g2_g']), r1(p['g2_be']))

    return hn, he, hg


# speedup vs baseline: 1.2020x; 1.0006x over previous
import functools

import numpy as np

import jax
import jax.numpy as jnp
from jax import lax
from jax.experimental import pallas as pl
from jax.experimental.pallas import tpu as pltpu
from jax.experimental.pallas import tpu_sc as plsc

D = 128
B = 10
NP = 1000
EP = 32000
N = B * NP
E = B * EP
EPS = 1e-5

NC, NS = 2, 16
NW = NC * NS
EPW = E // NW
CH = 128
NCH = EPW // CH
REM = EPW - NCH * CH
CG = 64
NCG = EPW // CG
REMG = EPW - NCG * CG
NPAD = 10112
RPT = NPAD // NS

TE = 2000
GE = E // TE
TPG = EP // TE

_f32 = jnp.float32
_bf16 = jnp.bfloat16


def _softplus(x):
    return jnp.maximum(x, 0.0) + jnp.log(1.0 + jnp.exp(-jnp.abs(x)))


def _sc_gather(nf, src, dst, zc):
    mesh = plsc.VectorSubcoreMesh(core_axis_name="c", subcore_axis_name="s")

    @functools.partial(
        pl.kernel,
        out_type=(jax.ShapeDtypeStruct((E, D), _f32),
                  jax.ShapeDtypeStruct((E, D), _f32),
                  jax.ShapeDtypeStruct((NC, NPAD, D), _f32)),
        mesh=mesh,
        scratch_types=[
            pltpu.VMEM_SHARED((NPAD, D), _f32),
            pltpu.VMEM((CG,), jnp.int32), pltpu.VMEM((CG,), jnp.int32),
            pltpu.VMEM((CG,), jnp.int32), pltpu.VMEM((CG,), jnp.int32),
            pltpu.VMEM((CG, D), _f32), pltpu.VMEM((CG, D), _f32),
            pltpu.VMEM((CG, D), _f32), pltpu.VMEM((CG, D), _f32),
            pltpu.VMEM((CG, D), _f32),
            pltpu.VMEM((REMG,), jnp.int32), pltpu.VMEM((REMG,), jnp.int32),
            pltpu.VMEM((REMG, D), _f32), pltpu.VMEM((REMG, D), _f32),
            pltpu.SemaphoreType.DMA, pltpu.SemaphoreType.DMA,
            pltpu.SemaphoreType.DMA, pltpu.SemaphoreType.DMA,
            pltpu.SemaphoreType.DMA, pltpu.SemaphoreType.DMA,
        ],
    )
    def k(nf_h, src_h, dst_h, zc_h, gs_h, gd_h, cnt_h,
          csum_sh, sA, dA, sB, dB, srA, drA, srB, drB, ones,
          sidx2, didx2, srow2, drow2,
          semIA, semIB, semGA, semGB, semCA, semCB):
        c = lax.axis_index("c")
        t = lax.axis_index("s")
        base0 = (c * NS + t) * EPW

        pltpu.sync_copy(zc_h.at[pl.ds(t * RPT, RPT)],
                        csum_sh.at[pl.ds(t * RPT, RPT)])

        def fill(i, carry):
            ones[i // 8, pl.ds((i % 8) * 16, 16)] = jnp.ones((16,), _f32)
            return carry
        lax.fori_loop(0, CG * D // 16, fill, 0)

        plsc.subcore_barrier()

        pltpu.async_copy(src_h.at[pl.ds(base0, CG)], sA, semIA)
        pltpu.async_copy(dst_h.at[pl.ds(base0, CG)], dA, semIA)

        npair = NCG // 2

        def pair(m, carry):
            baseA = base0 + (2 * m) * CG
            baseB = baseA + CG
            baseN = jnp.where(m == npair - 1,
                              base0, base0 + (2 * m + 2) * CG)
            pltpu.make_async_copy(src_h.at[pl.ds(baseA, CG)], sA, semIA).wait()
            pltpu.make_async_copy(dst_h.at[pl.ds(baseA, CG)], dA, semIA).wait()
            cpA1 = pltpu.async_copy(nf_h.at[sA], srA, semGA)
            cpA2 = pltpu.async_copy(nf_h.at[dA], drA, semGA)
            pltpu.async_copy(src_h.at[pl.ds(baseB, CG)], sB, semIB)
            pltpu.async_copy(dst_h.at[pl.ds(baseB, CG)], dB, semIB)
            ccA = pltpu.async_copy(ones, csum_sh.at[dA], semCA, add=True)
            pltpu.make_async_copy(src_h.at[pl.ds(baseB, CG)], sB, semIB).wait()
            pltpu.make_async_copy(dst_h.at[pl.ds(baseB, CG)], dB, semIB).wait()
            cpB1 = pltpu.async_copy(nf_h.at[sB], srB, semGB)
            cpB2 = pltpu.async_copy(nf_h.at[dB], drB, semGB)
            ccB = pltpu.async_copy(ones, csum_sh.at[dB], semCB, add=True)
            cpA1.wait()
            pltpu.sync_copy(srA, gs_h.at[pl.ds(baseA, CG)])
            cpA2.wait()
            pltpu.sync_copy(drA, gd_h.at[pl.ds(baseA, CG)])
            cpB1.wait()
            pltpu.sync_copy(srB, gs_h.at[pl.ds(baseB, CG)])
            cpB2.wait()
            pltpu.sync_copy(drB, gd_h.at[pl.ds(baseB, CG)])
            ccA.wait()
            ccB.wait()
            pltpu.async_copy(src_h.at[pl.ds(baseN, CG)], sA, semIA)
            pltpu.async_copy(dst_h.at[pl.ds(baseN, CG)], dA, semIA)
            return carry

        lax.fori_loop(0, npair, pair, 0)

        pltpu.make_async_copy(src_h.at[pl.ds(base0, CG)], sA, semIA).wait()
        pltpu.make_async_copy(dst_h.at[pl.ds(base0, CG)], dA, semIA).wait()

        base = base0 + NCG * CG
        pltpu.sync_copy(src_h.at[pl.ds(base, REMG)], sidx2)
        pltpu.sync_copy(dst_h.at[pl.ds(base, REMG)], didx2)
        cp1 = pltpu.async_copy(nf_h.at[sidx2], srow2, semGA)
        cp2 = pltpu.async_copy(nf_h.at[didx2], drow2, semGB)
        pltpu.sync_copy(ones.at[0:REMG], csum_sh.at[didx2], add=True)
        cp1.wait()
        pltpu.sync_copy(srow2, gs_h.at[pl.ds(base, REMG)])
        cp2.wait()
        pltpu.sync_copy(drow2, gd_h.at[pl.ds(base, REMG)])

        plsc.subcore_barrier()
        pltpu.sync_copy(csum_sh.at[pl.ds(t * RPT, RPT)],
                        cnt_h.at[c, pl.ds(t * RPT, RPT)])

    return k(nf, src, dst, zc)


def _sc_scatter(hep, dst, zn):
    mesh = plsc.VectorSubcoreMesh(core_axis_name="c", subcore_axis_name="s")

    @functools.partial(
        pl.kernel,
        out_type=jax.ShapeDtypeStruct((NC, NPAD, D), _f32),
        mesh=mesh,
        scratch_types=[
            pltpu.VMEM_SHARED((NPAD, D), _f32),
            pltpu.VMEM((CH,), jnp.int32), pltpu.VMEM((CH,), jnp.int32),
            pltpu.VMEM((CH, D), _f32), pltpu.VMEM((CH, D), _f32),
            pltpu.VMEM((REM,), jnp.int32),
            pltpu.VMEM((REM, D), _f32),
            pltpu.SemaphoreType.DMA, pltpu.SemaphoreType.DMA,
            pltpu.SemaphoreType.DMA, pltpu.SemaphoreType.DMA,
        ],
    )
    def k(hep_h, dst_h, zn_h, parts_h, hsum_sh,
          dA, dB, rowsA, rowsB, didx2, rows2,
          semIA, semIB, semSA, semSB):
        c = lax.axis_index("c")
        t = lax.axis_index("s")

        pltpu.sync_copy(zn_h.at[pl.ds(t * RPT, RPT)],
                        hsum_sh.at[pl.ds(t * RPT, RPT)])
        plsc.subcore_barrier()

        base0 = (c * NS + t) * EPW
        npair = NCH // 2

        pltpu.async_copy(dst_h.at[pl.ds(base0, CH)], dA, semIA)
        pltpu.async_copy(hep_h.at[pl.ds(base0, CH)], rowsA, semIA)

        def pair(m, carry):
            baseA = base0 + (2 * m) * CH
            baseB = baseA + CH
            baseN = jnp.where(m == npair - 1,
                              base0, base0 + (2 * m + 2) * CH)
            pltpu.make_async_copy(dst_h.at[pl.ds(baseA, CH)], dA, semIA).wait()
            pltpu.make_async_copy(hep_h.at[pl.ds(baseA, CH)], rowsA, semIA).wait()
            scA = pltpu.async_copy(rowsA, hsum_sh.at[dA], semSA, add=True)
            pltpu.async_copy(dst_h.at[pl.ds(baseB, CH)], dB, semIB)
            pltpu.async_copy(hep_h.at[pl.ds(baseB, CH)], rowsB, semIB)
            pltpu.make_async_copy(dst_h.at[pl.ds(baseB, CH)], dB, semIB).wait()
            pltpu.make_async_copy(hep_h.at[pl.ds(baseB, CH)], rowsB, semIB).wait()
            scB = pltpu.async_copy(rowsB, hsum_sh.at[dB], semSB, add=True)
            scA.wait()
            scB.wait()
            pltpu.async_copy(dst_h.at[pl.ds(baseN, CH)], dA, semIA)
            pltpu.async_copy(hep_h.at[pl.ds(baseN, CH)], rowsA, semIA)
            return carry

        lax.fori_loop(0, npair, pair, 0)

        pltpu.make_async_copy(dst_h.at[pl.ds(base0, CH)], dA, semIA).wait()
        pltpu.make_async_copy(hep_h.at[pl.ds(base0, CH)], rowsA, semIA).wait()

        base = base0 + NCH * CH
        pltpu.sync_copy(dst_h.at[pl.ds(base, REM)], didx2)
        pltpu.sync_copy(hep_h.at[pl.ds(base, REM)], rows2)
        pltpu.sync_copy(rows2, hsum_sh.at[didx2], add=True)

        plsc.subcore_barrier()

        pltpu.sync_copy(hsum_sh.at[pl.ds(t * RPT, RPT)],
                        parts_h.at[c, pl.ds(t * RPT, RPT)])

    return k(hep, dst, zn)


def _edge1(gs, gd, ef, gf3, w1t, b1):
    def body(gs_r, gd_r, ef_r, gf_r, w_r, b_r, h1_r, st_r):
        i = pl.program_id(0)
        gfb = gf_r[0].astype(_bf16)
        x = jnp.concatenate(
            [gs_r[...].astype(_bf16), gd_r[...].astype(_bf16),
             ef_r[...].astype(_bf16),
             jnp.broadcast_to(gfb, (TE, D))], axis=1)
        h = jnp.dot(x, w_r[...], preferred_element_type=_f32) + b_r[...]
        h1_r[...] = h.astype(jnp.bfloat16)
        s = jnp.sum(h, axis=0, keepdims=True)
        q = jnp.sum(h * h, axis=0, keepdims=True)
        st = jnp.concatenate([s, q], axis=0)

        @pl.when(i == 0)
        def _():
            st_r[...] = jnp.zeros_like(st_r)
        st_r[...] += st

    return pl.pallas_call(
        body,
        grid=(GE,),
        in_specs=[
            pl.BlockSpec((TE, D), lambda i: (i, 0)),
            pl.BlockSpec((TE, D), lambda i: (i, 0)),
            pl.BlockSpec((TE, D), lambda i: (i, 0)),
            pl.BlockSpec((1, 1, D), lambda i: (i // TPG, 0, 0)),
            pl.BlockSpec((4 * D, 4 * D), lambda i: (0, 0)),
            pl.BlockSpec((1, 4 * D), lambda i: (0, 0)),
        ],
        out_specs=[
            pl.BlockSpec((TE, 4 * D), lambda i: (i, 0)),
            pl.BlockSpec((2, 4 * D), lambda i: (0, 0)),
        ],
        out_shape=[jax.ShapeDtypeStruct((E, 4 * D), jnp.bfloat16),
                   jax.ShapeDtypeStruct((2, 4 * D), _f32)],
    )(gs, gd, ef, gf3, w1t, b1)


def _edge2(h1, st1, g1, be1, w2t, b2):
    def body(h1_r, st_r, g_r, be_r, w_r, b_r, h2_r, st2_r):
        i = pl.program_id(0)
        mu = st_r[0:1, :] * (1.0 / E)
        var = st_r[1:2, :] * (1.0 / E) - mu * mu
        sc = g_r[...] * lax.rsqrt(var + EPS)
        sh = be_r[...] - sc * mu
        a = _softplus(h1_r[...].astype(_f32) * sc + sh)
        h2 = jnp.dot(a.astype(_bf16), w_r[...],
                     preferred_element_type=_f32) + b_r[...]
        h2_r[...] = h2.astype(_bf16)
        s = jnp.sum(h2, axis=0, keepdims=True)
        q = jnp.sum(h2 * h2, axis=0, keepdims=True)
        st = jnp.concatenate([s, q], axis=0)

        @pl.when(i == 0)
        def _():
            st2_r[...] = jnp.zeros_like(st2_r)
        st2_r[...] += st

    return pl.pallas_call(
        body,
        grid=(GE,),
        in_specs=[
            pl.BlockSpec((TE, 4 * D), lambda i: (i, 0)),
            pl.BlockSpec((2, 4 * D), lambda i: (0, 0)),
            pl.BlockSpec((1, 4 * D), lambda i: (0, 0)),
            pl.BlockSpec((1, 4 * D), lambda i: (0, 0)),
            pl.BlockSpec((4 * D, D), lambda i: (0, 0)),
            pl.BlockSpec((1, D), lambda i: (0, 0)),
        ],
        out_specs=[
            pl.BlockSpec((TE, D), lambda i: (i, 0)),
            pl.BlockSpec((2, D), lambda i: (0, 0)),
        ],
        out_shape=[jax.ShapeDtypeStruct((E, D), jnp.bfloat16),
                   jax.ShapeDtypeStruct((2, D), _f32)],
    )(h1, st1, g1, be1, w2t, b2)


def _edge3(h2, st2, g2, be2):
    def body(h2_r, st_r, g_r, be_r, he_r):
        mu = st_r[0:1, :] * (1.0 / E)
        var = st_r[1:2, :] * (1.0 / E) - mu * mu
        sc = g_r[...] * lax.rsqrt(var + EPS)
        sh = be_r[...] - sc * mu
        he_r[...] = _softplus(h2_r[...].astype(_f32) * sc + sh)

    return pl.pallas_call(
        body,
        grid=(GE,),
        in_specs=[
            pl.BlockSpec((TE, D), lambda i: (i, 0)),
            pl.BlockSpec((2, D), lambda i: (0, 0)),
            pl.BlockSpec((1, D), lambda i: (0, 0)),
            pl.BlockSpec((1, D), lambda i: (0, 0)),
        ],
        out_specs=pl.BlockSpec((TE, D), lambda i: (i, 0)),
        out_shape=jax.ShapeDtypeStruct((E, D), _f32),
    )(h2, st2, g2, be2)


def _nodeglob(nf, parts, cnt, gf, w3t, b3, g3, be3, w4t, b4, g4, be4,
              wg1t, bg1, gg1, beg1, wg2t, bg2, gg2, beg2):
    def bn_rows(h, g, be, n):
        mu = jnp.mean(h, axis=0, keepdims=True)
        var = jnp.mean(h * h, axis=0, keepdims=True) - mu * mu
        return _softplus(g * ((h - mu) * lax.rsqrt(var + EPS)) + be)

    def body(nf_r, p_r, c_r, gf_r, w3_r, b3_r, g3_r, be3_r,
             w4_r, b4_r, g4_r, be4_r, wg1_r, bg1_r, gg1_r, beg1_r,
             wg2_r, bg2_r, gg2_r, beg2_r, hn_r, hg_r):
        pv = p_r[...]
        hs = pv[0] + pv[1]
        cv = c_r[...]
        cnt_ = (cv[0] + cv[1])
        have = hs / jnp.maximum(cnt_, 1.0)
        gfv = gf_r[...]
        gn = jnp.broadcast_to(gfv[:, None, :], (B, NP, D)).reshape(N, D)
        x = jnp.concatenate([nf_r[...], have, gn], axis=1)
        h3 = jnp.dot(x, w3_r[...], preferred_element_type=_f32) + b3_r[...]
        hn1 = bn_rows(h3, g3_r[...], be3_r[...], N)
        h4 = jnp.dot(hn1, w4_r[...], preferred_element_type=_f32) + b4_r[...]
        hn = bn_rows(h4, g4_r[...], be4_r[...], N)
        hn_r[...] = hn
        gnode = jnp.mean(hn.reshape(B, NP, D), axis=1)
        gedge = jnp.mean(have.reshape(B, NP, D), axis=1)
        xg = jnp.concatenate([gnode, gedge, gfv], axis=1)
        hg1 = jnp.dot(xg, wg1_r[...], preferred_element_type=_f32) + bg1_r[...]
        hg1 = bn_rows(hg1, gg1_r[...], beg1_r[...], B)
        hg2 = jnp.dot(hg1, wg2_r[...], preferred_element_type=_f32) + bg2_r[...]
        hg_r[...] = bn_rows(hg2, gg2_r[...], beg2_r[...], B)

    return pl.pallas_call(
        body,
        out_shape=[jax.ShapeDtypeStruct((N, D), _f32),
                   jax.ShapeDtypeStruct((B, D), _f32)],
    )(nf, parts, cnt, gf, w3t, b3, g3, be3, w4t, b4, g4, be4,
      wg1t, bg1, gg1, beg1, wg2t, bg2, gg2, beg2)


def kernel(node_feats, edge_feats, global_feats, params, src, dst,
           node_gid, batch_num_nodes, batch_num_edges):
    p = params
    r1 = lambda v: v.reshape(1, -1)
    gf3 = global_feats.reshape(B, 1, D)

    gs, gd, cparts = _sc_gather(node_feats, src, dst,
                                jnp.zeros((NPAD, D), _f32))
    h1, st1 = _edge1(gs, gd, edge_feats, gf3,
                     p['e1_W'].T.astype(_bf16), r1(p['e1_b']))
    h2, st2 = _edge2(h1, st1, r1(p['e1_g']), r1(p['e1_be']),
                     p['e2_W'].T.astype(_bf16), r1(p['e2_b']))
    he = _edge3(h2, st2, r1(p['e2_g']), r1(p['e2_be']))

    parts = _sc_scatter(he, dst, jnp.zeros((NPAD, D), _f32))

    hn, hg = _nodeglob(
        node_feats, parts[:, :N], cparts[:, :N, 0:1], global_feats,
        p['n1_W'].T, r1(p['n1_b']), r1(p['n1_g']), r1(p['n1_be']),
        p['n2_W'].T, r1(p['n2_b']), r1(p['n2_g']), r1(p['n2_be']),
        p['g1_W'].T, r1(p['g1_b']), r1(p['g1_g']), r1(p['g1_be']),
        p['g2_W'].T, r1(p['g2_b']), r1(p['g2_g']), r1(p['g2_be']))

    return hn, he, hg
